# Initial kernel scaffold; baseline (speedup 1.0000x reference)
import jax
import jax.numpy as jnp
from jax.experimental import pallas as pl


def kernel(x, edge_index, edge_attr, batch, conv_w, conv_b, W1, b1, W2, b2, W3, b3, g1, be1, g2, be2, g3, be3, fc_w, fc_b):
    raise NotImplementedError("write your pallas kernel here")


# speedup vs baseline: 1.0022x; 1.0022x over previous
import jax
import jax.numpy as jnp
from jax.experimental import pallas as pl


def _fc_logsoftmax_kernel(pooled_ref, w_ref, b_ref, out_ref):
    logits = jnp.dot(pooled_ref[...], w_ref[...],
                     preferred_element_type=jnp.float32) + b_ref[...]
    m = jnp.max(logits, axis=1, keepdims=True)
    s = logits - m
    lse = jnp.log(jnp.sum(jnp.exp(s), axis=1, keepdims=True))
    out_ref[...] = s - lse


def _gcn(x, src, dst, w, W, b, n):
    xw = x @ W
    loop = jnp.arange(n, dtype=src.dtype)
    s = jnp.concatenate([src, loop])
    d = jnp.concatenate([dst, loop])
    ww = jnp.concatenate([w, jnp.ones((n,), dtype=w.dtype)])
    deg = jnp.zeros((n,), dtype=w.dtype).at[d].add(ww)
    dinv = jnp.where(deg > 0, jax.lax.rsqrt(jnp.maximum(deg, 1e-12)), 0.0)
    norm = dinv[s] * ww * dinv[d]
    out = jnp.zeros_like(xw).at[d].add(xw[s] * norm[:, None])
    return out + b


def _bn(x, g, b):
    m = jnp.mean(x, axis=0)
    v = jnp.var(x, axis=0)
    return (x - m) * jax.lax.rsqrt(v + 1e-5) * g + b


def kernel(x, edge_index, edge_attr, batch, conv_w, conv_b, W1, b1, W2, b2, W3, b3, g1, be1, g2, be2, g3, be3, fc_w, fc_b):
    n = x.shape[0]
    num_graphs = 1024
    src, dst = edge_index[0], edge_index[1]
    h = jax.lax.conv_general_dilated(
        x[:, None, :], conv_w, window_strides=(20,), padding="VALID",
        dimension_numbers=("NCH", "OIH", "NCH"))
    h = jax.nn.relu(h + conv_b[None, :, None])
    h = h.reshape(n, -1)
    h = jax.nn.relu(_bn(_gcn(h, src, dst, edge_attr, W1, b1, n), g1, be1))
    h = jax.nn.relu(_bn(_gcn(h, src, dst, edge_attr, W2, b2, n), g2, be2))
    h = jax.nn.relu(_bn(_gcn(h, src, dst, edge_attr, W3, b3, n), g3, be3))
    sums = jax.ops.segment_sum(h, batch, num_segments=num_graphs)
    cnt = jax.ops.segment_sum(jnp.ones((n,), dtype=h.dtype), batch,
                              num_segments=num_graphs)
    pooled = sums / jnp.maximum(cnt, 1.0)[:, None]
    out = pl.pallas_call(
        _fc_logsoftmax_kernel,
        out_shape=jax.ShapeDtypeStruct((num_graphs, 4), jnp.float32),
    )(pooled, fc_w, fc_b)
    return out


# speedup vs baseline: 8.1293x; 8.1115x over previous
import functools

import jax
import jax.numpy as jnp
from jax import lax
from jax.experimental import pallas as pl
from jax.experimental.pallas import tpu as pltpu
from jax.experimental.pallas import tpu_sc as plsc

N = 50000
E0 = 800000
EA = E0 + N
NUM_GRAPHS = 1024
NB = 4
BW = 12500
BP = 12512
ROWS = 6648
NCHUNK = ROWS // 8
NWORK = 32
CAP = 27648
STG = 2320
BM = 2000
NBLK = N // BM


def _i32(x):
    return x.astype(jnp.int32)


def _extract_lane(vec, lane_const, ii):
    sel = jnp.where(ii == lane_const, vec, jnp.zeros_like(vec))
    return jnp.sum(sel)


def _k1_body(src_hbm, dst_hbm, w_hbm, deg2, srcb, dstb, wb, nbat,
             in_src, in_dst, in_w, st_src, st_dst, st_w, zv, nbv,
             dsem, deg_sp):
    c = lax.axis_index("c")
    s = lax.axis_index("s")
    wid = s * 2 + c
    ii = lax.iota(jnp.int32, 16)
    zf = jnp.zeros((16,), jnp.float32)

    for i in range(200):
        zv[pl.ds(16 * i, 16)] = zf

    @pl.when(s < 15)
    def _():
        pltpu.sync_copy(zv.at[pl.ds(0, 3200)],
                        deg_sp.at[pl.ds(pl.multiple_of(s * 3200, 128), 3200)])

    @pl.when(s == 15)
    def _():
        pltpu.sync_copy(zv.at[pl.ds(0, 2000)], deg_sp.at[pl.ds(48000, 2000)])

    plsc.subcore_barrier()

    nck = (NCHUNK - wid + NWORK - 1) // NWORK

    def chunk_body(k, carry):
        off = list(carry[0:4])
        wr = list(carry[4:8])
        r0 = pl.multiple_of((wid + NWORK * k) * 8, 8)
        pltpu.sync_copy(src_hbm.at[pl.ds(r0, 8)], in_src)
        pltpu.sync_copy(dst_hbm.at[pl.ds(r0, 8)], in_dst)
        pltpu.sync_copy(w_hbm.at[pl.ds(r0, 8)], in_w)

        handles = []
        for r in range(8):
            handles.append(pltpu.async_copy(
                in_w.at[r], deg_sp.at[in_dst.at[r]], dsem, add=True))
        for h in handles:
            h.wait()

        def group_body(g, gc):
            goff = list(gc)
            r = g // 8
            kk = g - 8 * r
            d = in_dst[r, pl.ds(kk * 16, 16)].reshape((16,))
            sv = in_src[r, pl.ds(kk * 16, 16)].reshape((16,))
            wv = in_w[r, pl.ds(kk * 16, 16)].reshape((16,))
            bid = (_i32(d >= BW) + _i32(d >= 2 * BW) + _i32(d >= 3 * BW))
            dl = d - bid * BW
            for b in range(NB):
                m = bid == b
                plsc.store_compressed(
                    st_src.at[pl.ds(b * STG + goff[b], 16)], sv, mask=m)
                plsc.store_compressed(
                    st_dst.at[pl.ds(b * STG + goff[b], 16)], dl, mask=m)
                plsc.store_compressed(
                    st_w.at[pl.ds(b * STG + goff[b], 16)], wv, mask=m)
                goff[b] = goff[b] + jnp.sum(_i32(m))
            return tuple(goff)

        off = list(lax.fori_loop(0, 64, group_body, tuple(off)))

        for b in range(NB):
            do = off[b] >= 1024

            @pl.when(do)
            def _(b=b, wrb=wr[b]):
                o = pl.multiple_of(wrb * 1024, 128)
                pltpu.sync_copy(st_src.at[pl.ds(b * STG, 1024)],
                                srcb.at[wid, b, pl.ds(o, 1024)])
                pltpu.sync_copy(st_dst.at[pl.ds(b * STG, 1024)],
                                dstb.at[wid, b, pl.ds(o, 1024)])
                pltpu.sync_copy(st_w.at[pl.ds(b * STG, 1024)],
                                wb.at[wid, b, pl.ds(o, 1024)])
                for i in range(64):
                    o_hi = b * STG + 1024 + 16 * i
                    o_lo = b * STG + 16 * i
                    st_src[pl.ds(o_lo, 16)] = st_src[pl.ds(o_hi, 16)]
                    st_dst[pl.ds(o_lo, 16)] = st_dst[pl.ds(o_hi, 16)]
                    st_w[pl.ds(o_lo, 16)] = st_w[pl.ds(o_hi, 16)]

            di = _i32(do)
            wr[b] = wr[b] + di
            off[b] = off[b] - 1024 * di
        return tuple(off) + tuple(wr)

    carry = lax.fori_loop(0, nck, chunk_body, (0, 0, 0, 0, 0, 0, 0, 0))
    off = list(carry[0:4])
    wr = list(carry[4:8])

    nbvec = jnp.zeros((16,), jnp.int32)
    for b in range(NB):
        for i in range(8):
            st_src[pl.ds(b * STG + off[b] + 16 * i, 16)] = ii
            st_dst[pl.ds(b * STG + off[b] + 16 * i, 16)] = ii
            st_w[pl.ds(b * STG + off[b] + 16 * i, 16)] = zf
        nblk = (off[b] + 127) // 128
        for i in range(8):
            @pl.when(i < nblk)
            def _(b=b, i=i, wrb=wr[b]):
                o = pl.multiple_of(wrb * 1024 + 128 * i, 128)
                pltpu.sync_copy(st_src.at[pl.ds(b * STG + 128 * i, 128)],
                                srcb.at[wid, b, pl.ds(o, 128)])
                pltpu.sync_copy(st_dst.at[pl.ds(b * STG + 128 * i, 128)],
                                dstb.at[wid, b, pl.ds(o, 128)])
                pltpu.sync_copy(st_w.at[pl.ds(b * STG + 128 * i, 128)],
                                wb.at[wid, b, pl.ds(o, 128)])
        tot = wr[b] * 8 + nblk
        nbvec = jnp.where(ii == b, jnp.full((16,), tot, jnp.int32), nbvec)
    nbv[...] = nbvec
    pltpu.sync_copy(nbv, nbat.at[wid])

    plsc.subcore_barrier()

    @pl.when(s < 15)
    def _():
        o = pl.multiple_of(s * 3200, 128)
        pltpu.sync_copy(deg_sp.at[pl.ds(o, 3200)], deg2.at[c, pl.ds(o, 3200)])

    @pl.when(s == 15)
    def _():
        pltpu.sync_copy(deg_sp.at[pl.ds(48000, 2000)],
                        deg2.at[c, pl.ds(48000, 2000)])


def _k1(src2d, dst2d, w2d):
    mesh = plsc.VectorSubcoreMesh(core_axis_name="c", subcore_axis_name="s")
    f = functools.partial(
        pl.kernel,
        mesh=mesh,
        compiler_params=pltpu.CompilerParams(
            needs_layout_passes=False, use_tc_tiling_on_sc=False),
        out_type=(
            jax.ShapeDtypeStruct((2, N), jnp.float32),
            jax.ShapeDtypeStruct((NWORK, NB, CAP), jnp.int32),
            jax.ShapeDtypeStruct((NWORK, NB, CAP), jnp.int32),
            jax.ShapeDtypeStruct((NWORK, NB, CAP), jnp.float32),
            jax.ShapeDtypeStruct((NWORK, 16), jnp.int32),
        ),
        scratch_types=[
            pltpu.VMEM((8, 128), jnp.int32),
            pltpu.VMEM((8, 128), jnp.int32),
            pltpu.VMEM((8, 128), jnp.float32),
            pltpu.VMEM((NB * STG,), jnp.int32),
            pltpu.VMEM((NB * STG,), jnp.int32),
            pltpu.VMEM((NB * STG,), jnp.float32),
            pltpu.VMEM((3200,), jnp.float32),
            pltpu.VMEM((16,), jnp.int32),
            pltpu.SemaphoreType.DMA,
            pltpu.VMEM_SHARED((N,), jnp.float32),
        ],
    )(_k1_body)
    return f(src2d, dst2d, w2d)


def _k2_body(deg2, srcb, dstb, wb, nbat, wnb,
             da, db, dv, sidx, didx, dgi, wv, dsv, ddv, wn, nbv,
             gsem, dinv_sp):
    c = lax.axis_index("c")
    s = lax.axis_index("s")
    wid = s * 2 + c
    ii = lax.iota(jnp.int32, 16)

    def newton(nv):
        half = jnp.full((16,), 0.5, jnp.float32)
        threeh = jnp.full((16,), 1.5, jnp.float32)
        magic = jnp.full((16,), 0x5f3759df, jnp.int32)
        for i in range(nv):
            x = da[pl.ds(16 * i, 16)] + db[pl.ds(16 * i, 16)]
            bits = plsc.bitcast(x, jnp.int32)
            y = plsc.bitcast(magic - lax.shift_right_logical(bits, jnp.full((16,), 1, jnp.int32)),
                             jnp.float32)
            for _ in range(3):
                y = y * (threeh - half * x * y * y)
            dv[pl.ds(16 * i, 16)] = y

    for half in range(2):
        sl = 2 * s + half

        @pl.when(sl < 31)
        def _(sl=sl):
            r0 = pl.multiple_of(sl * 1568, 8)
            pltpu.sync_copy(deg2.at[0, pl.ds(r0, 1568)], da)
            pltpu.sync_copy(deg2.at[1, pl.ds(r0, 1568)], db)

        @pl.when(sl == 31)
        def _():
            pltpu.sync_copy(deg2.at[0, pl.ds(48608, 1392)],
                            da.at[pl.ds(0, 1392)])
            pltpu.sync_copy(deg2.at[1, pl.ds(48608, 1392)],
                            db.at[pl.ds(0, 1392)])

        newton(98)

        @pl.when(sl < 31)
        def _(sl=sl):
            r0 = pl.multiple_of(sl * 1568, 8)
            pltpu.sync_copy(dv, dinv_sp.at[pl.ds(r0, 1568)])

        @pl.when(sl == 31)
        def _():
            pltpu.sync_copy(dv.at[pl.ds(0, 1392)],
                            dinv_sp.at[pl.ds(48608, 1392)])

    plsc.subcore_barrier()

    pltpu.sync_copy(nbat.at[wid], nbv)
    nbvec = nbv[...]
    for b in range(NB):
        nb_b = _extract_lane(nbvec, b, ii)

        def batch_body(k, _, b=b):
            o = pl.multiple_of(k * 128, 128)
            pltpu.sync_copy(srcb.at[wid, b, pl.ds(o, 128)], sidx)
            pltpu.sync_copy(dstb.at[wid, b, pl.ds(o, 128)], didx)
            pltpu.sync_copy(wb.at[wid, b, pl.ds(o, 128)], wv)
            base = jnp.full((16,), b * BW, jnp.int32)
            for j in range(8):
                dgi[pl.ds(16 * j, 16)] = didx[pl.ds(16 * j, 16)] + base
            pltpu.async_copy(dinv_sp.at[sidx], dsv, gsem).wait()
            pltpu.async_copy(dinv_sp.at[dgi], ddv, gsem).wait()
            for j in range(8):
                wn[pl.ds(16 * j, 16)] = (wv[pl.ds(16 * j, 16)]
                                         * dsv[pl.ds(16 * j, 16)]
                                         * ddv[pl.ds(16 * j, 16)])
            pltpu.sync_copy(wn, wnb.at[wid, b, pl.ds(o, 128)])
            return 0

        lax.fori_loop(0, nb_b, batch_body, 0)


def _k2(deg2, srcb, dstb, wb, nbat):
    mesh = plsc.VectorSubcoreMesh(core_axis_name="c", subcore_axis_name="s")
    f = functools.partial(
        pl.kernel,
        mesh=mesh,
        compiler_params=pltpu.CompilerParams(
            needs_layout_passes=False, use_tc_tiling_on_sc=False),
        out_type=(jax.ShapeDtypeStruct((NWORK, NB, CAP), jnp.float32),),
        scratch_types=[
            pltpu.VMEM((1568,), jnp.float32),
            pltpu.VMEM((1568,), jnp.float32),
            pltpu.VMEM((1568,), jnp.float32),
            pltpu.VMEM((128,), jnp.int32),
            pltpu.VMEM((128,), jnp.int32),
            pltpu.VMEM((128,), jnp.int32),
            pltpu.VMEM((128,), jnp.float32),
            pltpu.VMEM((128,), jnp.float32),
            pltpu.VMEM((128,), jnp.float32),
            pltpu.VMEM((128,), jnp.float32),
            pltpu.VMEM((16,), jnp.int32),
            pltpu.SemaphoreType.DMA,
            pltpu.VMEM_SHARED((N,), jnp.float32),
        ],
    )(_k2_body)
    return f(deg2, srcb, dstb, wb, nbat)


def _make_prop(F):
    FC = F // 16

    def body(xw, srcb, dstb, wnb, nbat, S,
             sidx, didx, wv, rows, nbv, zb, gsem, ssem, S_sp):
        c = lax.axis_index("c")
        s = lax.axis_index("s")
        ii = lax.iota(jnp.int32, 16)
        zf = jnp.zeros((16,), jnp.float32)

        for r in range(46):
            for j in range(FC):
                zb[r, pl.ds(16 * j, 16)] = zf

        for phase in range(2):
            b = 2 * c + phase
            plsc.subcore_barrier()
            for i in range(17):
                pltpu.sync_copy(zb, S_sp.at[pl.ds((s * 17 + i) * 46, 46)])
            plsc.subcore_barrier()

            for t_off in range(2):
                t = 2 * s + t_off
                pltpu.sync_copy(nbat.at[t], nbv)
                nb_b = _extract_lane(nbv[...], b, ii)

                def batch_body(k, _, t=t, b=b):
                    o = pl.multiple_of(k * 128, 128)
                    pltpu.sync_copy(srcb.at[t, b, pl.ds(o, 128)], sidx)
                    pltpu.sync_copy(dstb.at[t, b, pl.ds(o, 128)], didx)
                    pltpu.sync_copy(wnb.at[t, b, pl.ds(o, 128)], wv)
                    pltpu.async_copy(xw.at[sidx], rows, gsem).wait()

                    def e_outer(j, _2):
                        for i_ in range(8):
                            e = j * 8 + i_
                            wspl = plsc.load_gather(
                                wv, [jnp.full((16,), e, jnp.int32)])
                            for fc in range(FC):
                                v = rows[e, pl.ds(16 * fc, 16)].reshape(
                                    (16,)) * wspl
                                rows[e, pl.ds(16 * fc, 16)] = v
                        return 0

                    lax.fori_loop(0, 16, e_outer, 0)
                    pltpu.async_copy(rows, S_sp.at[didx], ssem,
                                     add=True).wait()
                    return 0

                lax.fori_loop(0, nb_b, batch_body, 0)

            plsc.subcore_barrier()
            r0 = 782 * s

            @pl.when(s < 15)
            def _(b=b, r0=r0):
                pltpu.sync_copy(S_sp.at[pl.ds(r0, 782)],
                                S.at[pl.ds(b * BW + r0, 782)])

            @pl.when(s == 15)
            def _(b=b):
                pltpu.sync_copy(S_sp.at[pl.ds(11730, 770)],
                                S.at[pl.ds(b * BW + 11730, 770)])

    mesh = plsc.VectorSubcoreMesh(core_axis_name="c", subcore_axis_name="s")

    def run(xw, srcb, dstb, wnb, nbat):
        f = functools.partial(
            pl.kernel,
            mesh=mesh,
            compiler_params=pltpu.CompilerParams(
                needs_layout_passes=False, use_tc_tiling_on_sc=False),
            out_type=(jax.ShapeDtypeStruct((N, F), jnp.float32),),
            scratch_types=[
                pltpu.VMEM((128,), jnp.int32),
                pltpu.VMEM((128,), jnp.int32),
                pltpu.VMEM((128,), jnp.float32),
                pltpu.VMEM((128, F), jnp.float32),
                pltpu.VMEM((16,), jnp.int32),
                pltpu.VMEM((46, F), jnp.float32),
                pltpu.SemaphoreType.DMA,
                pltpu.SemaphoreType.DMA,
                pltpu.VMEM_SHARED((BP, F), jnp.float32),
            ],
        )(body)
        res = f(xw, srcb, dstb, wnb, nbat)
        return res[0] if isinstance(res, (tuple, list)) else res

    return run


def _pool_body(S3, ac, batch, psum2, pcnt2,
               tb, bidx, tb48, bidx48, ones112, ones48, acv, zb, zc,
               gsem, psum_sp, pcnt_sp):
    c = lax.axis_index("c")
    s = lax.axis_index("s")
    wid = s * 2 + c
    zf = jnp.zeros((16,), jnp.float32)
    onef = jnp.full((16,), 1.0, jnp.float32)

    for r in range(64):
        for j in range(4):
            zb[r, pl.ds(16 * j, 16)] = zf
    for i in range(4):
        zc[pl.ds(16 * i, 16)] = zf
    for i in range(7):
        ones112[pl.ds(16 * i, 16)] = onef
    for i in range(3):
        ones48[pl.ds(16 * i, 16)] = onef

    pltpu.sync_copy(ac, acv)
    a_l = [acv[0, pl.ds(16 * j, 16)].reshape((16,)) for j in range(4)]
    c_l = [acv[1, pl.ds(16 * j, 16)].reshape((16,)) for j in range(4)]

    pltpu.sync_copy(zb, psum_sp.at[pl.ds(s * 64, 64)])
    pltpu.sync_copy(zc, pcnt_sp.at[pl.ds(s * 64, 64)])
    plsc.subcore_barrier()

    def do_chunk(buf, bx, ones, nrows, r0):
        pltpu.sync_copy(S3.at[pl.ds(r0, nrows)], buf)
        pltpu.sync_copy(batch.at[pl.ds(r0, nrows)], bx)

        def row_body(r8, _):
            for i_ in range(8):
                r = r8 * 8 + i_
                for j in range(4):
                    v = buf[r, pl.ds(16 * j, 16)].reshape((16,))
                    z = jnp.maximum(v * a_l[j] + c_l[j], 0.0)
                    buf[r, pl.ds(16 * j, 16)] = z
            return 0

        lax.fori_loop(0, nrows // 8, row_body, 0)
        pltpu.async_copy(buf, psum_sp.at[bx], gsem, add=True).wait()
        pltpu.async_copy(ones, pcnt_sp.at[bx], gsem, add=True).wait()

    @pl.when(wid < 31)
    def _():
        def chunk_loop(k, _):
            r0 = pl.multiple_of(wid * 1568 + k * 112, 8)
            do_chunk(tb, bidx, ones112, 112, r0)
            return 0
        lax.fori_loop(0, 14, chunk_loop, 0)

    @pl.when(wid == 31)
    def _():
        def chunk_loop(k, _):
            r0 = pl.multiple_of(48608 + k * 112, 8)
            do_chunk(tb, bidx, ones112, 112, r0)
            return 0
        lax.fori_loop(0, 12, chunk_loop, 0)
        do_chunk(tb48, bidx48, ones48, 48, 49952)

    plsc.subcore_barrier()
    pltpu.sync_copy(psum_sp.at[pl.ds(s * 64, 64)],
                    psum2.at[c, pl.ds(s * 64, 64)])
    pltpu.sync_copy(pcnt_sp.at[pl.ds(s * 64, 64)],
                    pcnt2.at[c, pl.ds(s * 64, 64)])


def _pool(S3, ac, batch):
    mesh = plsc.VectorSubcoreMesh(core_axis_name="c", subcore_axis_name="s")
    f = functools.partial(
        pl.kernel,
        mesh=mesh,
        compiler_params=pltpu.CompilerParams(
            needs_layout_passes=False, use_tc_tiling_on_sc=False),
        out_type=(
            jax.ShapeDtypeStruct((2, NUM_GRAPHS, 64), jnp.float32),
            jax.ShapeDtypeStruct((2, NUM_GRAPHS), jnp.float32),
        ),
        scratch_types=[
            pltpu.VMEM((112, 64), jnp.float32),
            pltpu.VMEM((112,), jnp.int32),
            pltpu.VMEM((48, 64), jnp.float32),
            pltpu.VMEM((48,), jnp.int32),
            pltpu.VMEM((112,), jnp.float32),
            pltpu.VMEM((48,), jnp.float32),
            pltpu.VMEM((2, 64), jnp.float32),
            pltpu.VMEM((64, 64), jnp.float32),
            pltpu.VMEM((64,), jnp.float32),
            pltpu.SemaphoreType.DMA,
            pltpu.VMEM_SHARED((NUM_GRAPHS, 64), jnp.float32),
            pltpu.VMEM_SHARED((NUM_GRAPHS,), jnp.float32),
        ],
    )(_pool_body)
    return f(S3, ac, batch)


def _tc1_kernel(x_ref, wc_ref, cb_ref, w1_ref, out_ref):
    h = jnp.maximum(
        jnp.dot(x_ref[...], wc_ref[...],
                preferred_element_type=jnp.float32) + cb_ref[...], 0.0)
    out_ref[...] = jnp.dot(h, w1_ref[...], preferred_element_type=jnp.float32)


def _tc1(x, wc, cb, w1):
    return pl.pallas_call(
        _tc1_kernel,
        grid=(NBLK,),
        in_specs=[
            pl.BlockSpec((BM, 200), lambda i: (i, 0)),
            pl.BlockSpec((200, 192), lambda i: (0, 0)),
            pl.BlockSpec((1, 192), lambda i: (0, 0)),
            pl.BlockSpec((192, 128), lambda i: (0, 0)),
        ],
        out_specs=pl.BlockSpec((BM, 128), lambda i: (i, 0)),
        out_shape=jax.ShapeDtypeStruct((N, 128), jnp.float32),
    )(x, wc, cb, w1)


def _make_tcb(F):
    def kern(s_ref, b_ref, g_ref, be_ref, out_ref, acc_ref):
        i = pl.program_id(0)

        @pl.when(i == 0)
        def _():
            acc_ref[...] = jnp.zeros_like(acc_ref)

        t = s_ref[...] + b_ref[...]
        acc_ref[0:1, :] += jnp.sum(t, axis=0, keepdims=True)
        acc_ref[1:2, :] += jnp.sum(t * t, axis=0, keepdims=True)

        @pl.when(i == NBLK - 1)
        def _():
            mu = acc_ref[0:1, :] / N
            var = acc_ref[1:2, :] / N - mu * mu
            rs = lax.rsqrt(var + 1e-5)
            a = rs * g_ref[...]
            out_ref[0:1, :] = a
            out_ref[1:2, :] = (b_ref[...] - mu) * a + be_ref[...]

    def run(S, b, g, be):
        return pl.pallas_call(
            kern,
            grid=(NBLK,),
            in_specs=[
                pl.BlockSpec((BM, F), lambda i: (i, 0)),
                pl.BlockSpec((1, F), lambda i: (0, 0)),
                pl.BlockSpec((1, F), lambda i: (0, 0)),
                pl.BlockSpec((1, F), lambda i: (0, 0)),
            ],
            out_specs=pl.BlockSpec((2, F), lambda i: (0, 0)),
            out_shape=jax.ShapeDtypeStruct((2, F), jnp.float32),
            scratch_shapes=[pltpu.VMEM((2, F), jnp.float32)],
        )(S, b.reshape(1, F), g.reshape(1, F), be.reshape(1, F))

    return run


def _make_tc23(F, FO):
    def kern(s_ref, ac_ref, w_ref, out_ref):
        z = jnp.maximum(s_ref[...] * ac_ref[0:1, :] + ac_ref[1:2, :], 0.0)
        out_ref[...] = jnp.dot(z, w_ref[...],
                               preferred_element_type=jnp.float32)

    def run(S, ac, w):
        return pl.pallas_call(
            kern,
            grid=(NBLK,),
            in_specs=[
                pl.BlockSpec((BM, F), lambda i: (i, 0)),
                pl.BlockSpec((2, F), lambda i: (0, 0)),
                pl.BlockSpec((F, FO), lambda i: (0, 0)),
            ],
            out_specs=pl.BlockSpec((BM, FO), lambda i: (i, 0)),
            out_shape=jax.ShapeDtypeStruct((N, FO), jnp.float32),
        )(S, ac, w)

    return run


def _tcf_kernel(ps_ref, pc_ref, fw_ref, fb_ref, out_ref):
    ps = ps_ref[0] + ps_ref[1]
    cnt = pc_ref[0] + pc_ref[1]
    pooled = ps / jnp.maximum(cnt, 1.0)[:, None]
    logits = jnp.dot(pooled, fw_ref[...],
                     preferred_element_type=jnp.float32) + fb_ref[...]
    col = lax.broadcasted_iota(jnp.int32, logits.shape, 1)
    logits = jnp.where(col < 4, logits, -1e30)
    m = jnp.max(logits, axis=1, keepdims=True)
    sh = logits - m
    lse = jnp.log(jnp.sum(jnp.exp(sh), axis=1, keepdims=True))
    out_ref[...] = sh - lse


def _tcf(psum2, pcnt2, fw, fb):
    return pl.pallas_call(
        _tcf_kernel,
        out_shape=jax.ShapeDtypeStruct((NUM_GRAPHS, 128), jnp.float32),
    )(psum2, pcnt2, fw, fb)


_DBG = "full"


def kernel(x, edge_index, edge_attr, batch, conv_w, conv_b, W1, b1, W2, b2, W3, b3, g1, be1, g2, be2, g3, be3, fc_w, fc_b):
    loop = jnp.arange(N, dtype=jnp.int32)
    pad = ROWS * 128 - EA
    srcA = jnp.concatenate(
        [edge_index[0], loop, jnp.zeros((pad,), jnp.int32)]).reshape(ROWS, 128)
    dstA = jnp.concatenate(
        [edge_index[1], loop, jnp.zeros((pad,), jnp.int32)]).reshape(ROWS, 128)
    wA = jnp.concatenate(
        [edge_attr, jnp.ones((N,), jnp.float32),
         jnp.zeros((pad,), jnp.float32)]).reshape(ROWS, 128)

    deg2, srcb, dstb, wb, nbat = _k1(srcA, dstA, wA)
    wnb = _k2(deg2, srcb, dstb, wb, nbat)
    if isinstance(wnb, (tuple, list)):
        wnb = wnb[0]

    wct = conv_w[:, 0, :].T
    wc = jnp.zeros((200, 32, 6), jnp.float32)
    for r in range(6):
        wc = wc.at[20 * r:20 * r + 100, :, r].set(wct)
    wc = wc.reshape(200, 192)
    cb = jnp.repeat(conv_b, 6).reshape(1, 192)

    p128 = _make_prop(128)
    p64 = _make_prop(64)
    tcb128 = _make_tcb(128)
    tcb64 = _make_tcb(64)

    def prop_jnp(q):
        cnt = nbat[:, :NB] * 128
        ar = jnp.arange(CAP)[None, None, :]
        valid = ar < cnt[:, :, None]
        w_eff = jnp.where(valid, wnb, 0.0).reshape(-1)
        src_eff = jnp.where(valid, srcb, 0).reshape(-1)
        dg = jnp.clip(dstb, 0, BW - 1) + (
            jnp.arange(NB, dtype=jnp.int32)[None, :, None] * BW)
        dst_eff = jnp.where(valid, dg, 0).reshape(-1)
        msg = q[src_eff] * w_eff[:, None]
        return jnp.zeros_like(q).at[dst_eff].add(msg)

    def bn_jnp(t, g, be):
        mu = jnp.mean(t, axis=0)
        va = jnp.var(t, axis=0)
        return (t - mu) * lax.rsqrt(va + 1e-5) * g + be

    if _DBG in ("k12", "prop"):
        prop = prop_jnp if _DBG == "k12" else (
            lambda q, F=None: None)
        xw1j = jnp.maximum(x @ wc + cb, 0.0) @ W1
        if _DBG == "k12":
            S1j = prop_jnp(xw1j)
        else:
            S1j = p128(xw1j, srcb, dstb, wnb, nbat)
        h = jnp.maximum(bn_jnp(S1j + b1, g1, be1), 0.0)
        xw2j = h @ W2
        S2j = prop_jnp(xw2j) if _DBG == "k12" else p128(
            xw2j, srcb, dstb, wnb, nbat)
        h = jnp.maximum(bn_jnp(S2j + b2, g2, be2), 0.0)
        xw3j = h @ W3
        S3j = prop_jnp(xw3j) if _DBG == "k12" else p64(
            xw3j, srcb, dstb, wnb, nbat)
        h = jnp.maximum(bn_jnp(S3j + b3, g3, be3), 0.0)
        sums = jax.ops.segment_sum(h, batch, num_segments=NUM_GRAPHS)
        cntg = jax.ops.segment_sum(jnp.ones((N,), jnp.float32), batch,
                                   num_segments=NUM_GRAPHS)
        pooled = sums / jnp.maximum(cntg, 1.0)[:, None]
        logits = pooled @ fc_w + fc_b
        return jax.nn.log_softmax(logits, axis=1)

    xw1 = _tc1(x, wc, cb, W1)
    S1 = prop_jnp(xw1) if _DBG == "tc" else p128(xw1, srcb, dstb, wnb, nbat)
    ac1 = tcb128(S1, b1, g1, be1)
    xw2 = _make_tc23(128, 128)(S1, ac1, W2)
    S2 = prop_jnp(xw2) if _DBG == "tc" else p128(xw2, srcb, dstb, wnb, nbat)
    ac2 = tcb128(S2, b2, g2, be2)
    xw3 = _make_tc23(128, 64)(S2, ac2, W3)
    S3 = prop_jnp(xw3) if _DBG == "tc" else p64(xw3, srcb, dstb, wnb, nbat)
    ac3 = tcb64(S3, b3, g3, be3)
    psum2, pcnt2 = _pool(S3, ac3, batch)

    fw = jnp.zeros((64, 128), jnp.float32).at[:, :4].set(fc_w)
    fb = jnp.zeros((1, 128), jnp.float32).at[0, :4].set(fc_b)
    out = _tcf(psum2, pcnt2, fw, fb)
    return out[:, :4]


# speedup vs baseline: 11.2246x; 1.3808x over previous
import functools

import jax
import jax.numpy as jnp
from jax import lax
from jax.experimental import pallas as pl
from jax.experimental.pallas import tpu as pltpu
from jax.experimental.pallas import tpu_sc as plsc

N = 50000
E0 = 800000
EA = E0 + N
NUM_GRAPHS = 1024
NB = 8
BW = 6250
BP = 6272
ROWS = 6648
NCHUNK = ROWS // 8
NWORK = 32
CAP = 27648
STG = 2320
BM = 2000
NBLK = N // BM


def _i32(x):
    return x.astype(jnp.int32)


def _extract_lane(vec, lane_const, ii):
    sel = jnp.where(ii == lane_const, vec, jnp.zeros_like(vec))
    return jnp.sum(sel)


def _k1_body(src_hbm, dst_hbm, w_hbm, deg2, srcb, dstb, wb, nbat,
             in_src, in_dst, in_w, st_src, st_dst, st_w, zv, nbv,
             dsem, deg_sp):
    c = lax.axis_index("c")
    s = lax.axis_index("s")
    wid = s * 2 + c
    ii = lax.iota(jnp.int32, 16)
    zf = jnp.zeros((16,), jnp.float32)

    for i in range(200):
        zv[pl.ds(16 * i, 16)] = zf

    @pl.when(s < 15)
    def _():
        pltpu.sync_copy(zv.at[pl.ds(0, 3200)],
                        deg_sp.at[pl.ds(pl.multiple_of(s * 3200, 128), 3200)])

    @pl.when(s == 15)
    def _():
        pltpu.sync_copy(zv.at[pl.ds(0, 2000)], deg_sp.at[pl.ds(48000, 2000)])

    plsc.subcore_barrier()

    nck = (NCHUNK - wid + NWORK - 1) // NWORK

    def chunk_body(k, carry):
        off = list(carry[0:NB])
        wr = list(carry[NB:2 * NB])
        r0 = pl.multiple_of((wid + NWORK * k) * 8, 8)
        pltpu.sync_copy(src_hbm.at[pl.ds(r0, 8)], in_src)
        pltpu.sync_copy(dst_hbm.at[pl.ds(r0, 8)], in_dst)
        pltpu.sync_copy(w_hbm.at[pl.ds(r0, 8)], in_w)

        handles = []
        for r in range(8):
            handles.append(pltpu.async_copy(
                in_w.at[r], deg_sp.at[in_dst.at[r]], dsem, add=True))
        for h in handles:
            h.wait()

        def group_body(g, gc):
            goff = list(gc)
            r = g // 8
            kk = g - 8 * r
            d = in_dst[r, pl.ds(kk * 16, 16)].reshape((16,))
            sv = in_src[r, pl.ds(kk * 16, 16)].reshape((16,))
            wv = in_w[r, pl.ds(kk * 16, 16)].reshape((16,))
            bid = sum(_i32(d >= kk * BW) for kk in range(1, NB))
            dl = d - bid * BW
            for b in range(NB):
                m = bid == b
                plsc.store_compressed(
                    st_src.at[pl.ds(b * STG + goff[b], 16)], sv, mask=m)
                plsc.store_compressed(
                    st_dst.at[pl.ds(b * STG + goff[b], 16)], dl, mask=m)
                plsc.store_compressed(
                    st_w.at[pl.ds(b * STG + goff[b], 16)], wv, mask=m)
                goff[b] = goff[b] + jnp.sum(_i32(m))
            return tuple(goff)

        off = list(lax.fori_loop(0, 64, group_body, tuple(off)))

        for b in range(NB):
            do = off[b] >= 1024

            @pl.when(do)
            def _(b=b, wrb=wr[b]):
                o = pl.multiple_of(wrb * 1024, 128)
                pltpu.sync_copy(st_src.at[pl.ds(b * STG, 1024)],
                                srcb.at[wid, b, pl.ds(o, 1024)])
                pltpu.sync_copy(st_dst.at[pl.ds(b * STG, 1024)],
                                dstb.at[wid, b, pl.ds(o, 1024)])
                pltpu.sync_copy(st_w.at[pl.ds(b * STG, 1024)],
                                wb.at[wid, b, pl.ds(o, 1024)])
                for i in range(64):
                    o_hi = b * STG + 1024 + 16 * i
                    o_lo = b * STG + 16 * i
                    st_src[pl.ds(o_lo, 16)] = st_src[pl.ds(o_hi, 16)]
                    st_dst[pl.ds(o_lo, 16)] = st_dst[pl.ds(o_hi, 16)]
                    st_w[pl.ds(o_lo, 16)] = st_w[pl.ds(o_hi, 16)]

            di = _i32(do)
            wr[b] = wr[b] + di
            off[b] = off[b] - 1024 * di
        return tuple(off) + tuple(wr)

    carry = lax.fori_loop(0, nck, chunk_body, (0,) * (2 * NB))
    off = list(carry[0:NB])
    wr = list(carry[NB:2 * NB])

    nbvec = jnp.zeros((16,), jnp.int32)
    for b in range(NB):
        for i in range(8):
            st_src[pl.ds(b * STG + off[b] + 16 * i, 16)] = ii
            st_dst[pl.ds(b * STG + off[b] + 16 * i, 16)] = ii
            st_w[pl.ds(b * STG + off[b] + 16 * i, 16)] = zf
        nblk = (off[b] + 127) // 128
        for i in range(8):
            @pl.when(i < nblk)
            def _(b=b, i=i, wrb=wr[b]):
                o = pl.multiple_of(wrb * 1024 + 128 * i, 128)
                pltpu.sync_copy(st_src.at[pl.ds(b * STG + 128 * i, 128)],
                                srcb.at[wid, b, pl.ds(o, 128)])
                pltpu.sync_copy(st_dst.at[pl.ds(b * STG + 128 * i, 128)],
                                dstb.at[wid, b, pl.ds(o, 128)])
                pltpu.sync_copy(st_w.at[pl.ds(b * STG + 128 * i, 128)],
                                wb.at[wid, b, pl.ds(o, 128)])
        tot = wr[b] * 8 + nblk
        nbvec = jnp.where(ii == b, jnp.full((16,), tot, jnp.int32), nbvec)
    nbv[...] = nbvec
    pltpu.sync_copy(nbv, nbat.at[wid])

    plsc.subcore_barrier()

    @pl.when(s < 15)
    def _():
        o = pl.multiple_of(s * 3200, 128)
        pltpu.sync_copy(deg_sp.at[pl.ds(o, 3200)], deg2.at[c, pl.ds(o, 3200)])

    @pl.when(s == 15)
    def _():
        pltpu.sync_copy(deg_sp.at[pl.ds(48000, 2000)],
                        deg2.at[c, pl.ds(48000, 2000)])


def _k1(src2d, dst2d, w2d):
    mesh = plsc.VectorSubcoreMesh(core_axis_name="c", subcore_axis_name="s")
    f = functools.partial(
        pl.kernel,
        mesh=mesh,
        compiler_params=pltpu.CompilerParams(
            needs_layout_passes=False, use_tc_tiling_on_sc=False),
        out_type=(
            jax.ShapeDtypeStruct((2, N), jnp.float32),
            jax.ShapeDtypeStruct((NWORK, NB, CAP), jnp.int32),
            jax.ShapeDtypeStruct((NWORK, NB, CAP), jnp.int32),
            jax.ShapeDtypeStruct((NWORK, NB, CAP), jnp.float32),
            jax.ShapeDtypeStruct((NWORK, 16), jnp.int32),
        ),
        scratch_types=[
            pltpu.VMEM((8, 128), jnp.int32),
            pltpu.VMEM((8, 128), jnp.int32),
            pltpu.VMEM((8, 128), jnp.float32),
            pltpu.VMEM((NB * STG,), jnp.int32),
            pltpu.VMEM((NB * STG,), jnp.int32),
            pltpu.VMEM((NB * STG,), jnp.float32),
            pltpu.VMEM((3200,), jnp.float32),
            pltpu.VMEM((16,), jnp.int32),
            pltpu.SemaphoreType.DMA,
            pltpu.VMEM_SHARED((N,), jnp.float32),
        ],
    )(_k1_body)
    return f(src2d, dst2d, w2d)


def _k2_body(deg2, srcb, dstb, wb, nbat, wnb,
             da, db, dv, sidx, didx, dgi, wv, dsv, ddv, wn, nbv,
             gsem, dinv_sp):
    c = lax.axis_index("c")
    s = lax.axis_index("s")
    wid = s * 2 + c
    ii = lax.iota(jnp.int32, 16)

    def newton(nv):
        half = jnp.full((16,), 0.5, jnp.float32)
        threeh = jnp.full((16,), 1.5, jnp.float32)
        magic = jnp.full((16,), 0x5f3759df, jnp.int32)
        for i in range(nv):
            x = da[pl.ds(16 * i, 16)] + db[pl.ds(16 * i, 16)]
            bits = plsc.bitcast(x, jnp.int32)
            y = plsc.bitcast(magic - lax.shift_right_logical(bits, jnp.full((16,), 1, jnp.int32)),
                             jnp.float32)
            for _ in range(3):
                y = y * (threeh - half * x * y * y)
            dv[pl.ds(16 * i, 16)] = y

    for half in range(2):
        sl = 2 * s + half

        @pl.when(sl < 31)
        def _(sl=sl):
            r0 = pl.multiple_of(sl * 1568, 8)
            pltpu.sync_copy(deg2.at[0, pl.ds(r0, 1568)], da)
            pltpu.sync_copy(deg2.at[1, pl.ds(r0, 1568)], db)

        @pl.when(sl == 31)
        def _():
            pltpu.sync_copy(deg2.at[0, pl.ds(48608, 1392)],
                            da.at[pl.ds(0, 1392)])
            pltpu.sync_copy(deg2.at[1, pl.ds(48608, 1392)],
                            db.at[pl.ds(0, 1392)])

        newton(98)

        @pl.when(sl < 31)
        def _(sl=sl):
            r0 = pl.multiple_of(sl * 1568, 8)
            pltpu.sync_copy(dv, dinv_sp.at[pl.ds(r0, 1568)])

        @pl.when(sl == 31)
        def _():
            pltpu.sync_copy(dv.at[pl.ds(0, 1392)],
                            dinv_sp.at[pl.ds(48608, 1392)])

    plsc.subcore_barrier()

    pltpu.sync_copy(nbat.at[wid], nbv)
    nbvec = nbv[...]
    for b in range(NB):
        nb_b = _extract_lane(nbvec, b, ii)

        def batch_body(k, _, b=b):
            o = pl.multiple_of(k * 128, 128)
            pltpu.sync_copy(srcb.at[wid, b, pl.ds(o, 128)], sidx)
            pltpu.sync_copy(dstb.at[wid, b, pl.ds(o, 128)], didx)
            pltpu.sync_copy(wb.at[wid, b, pl.ds(o, 128)], wv)
            base = jnp.full((16,), b * BW, jnp.int32)
            for j in range(8):
                dgi[pl.ds(16 * j, 16)] = didx[pl.ds(16 * j, 16)] + base
            pltpu.async_copy(dinv_sp.at[sidx], dsv, gsem).wait()
            pltpu.async_copy(dinv_sp.at[dgi], ddv, gsem).wait()
            for j in range(8):
                wn[pl.ds(16 * j, 16)] = (wv[pl.ds(16 * j, 16)]
                                         * dsv[pl.ds(16 * j, 16)]
                                         * ddv[pl.ds(16 * j, 16)])
            pltpu.sync_copy(wn, wnb.at[wid, b, pl.ds(o, 128)])
            return 0

        lax.fori_loop(0, nb_b, batch_body, 0)


def _k2(deg2, srcb, dstb, wb, nbat):
    mesh = plsc.VectorSubcoreMesh(core_axis_name="c", subcore_axis_name="s")
    f = functools.partial(
        pl.kernel,
        mesh=mesh,
        compiler_params=pltpu.CompilerParams(
            needs_layout_passes=False, use_tc_tiling_on_sc=False),
        out_type=(jax.ShapeDtypeStruct((NWORK, NB, CAP), jnp.float32),),
        scratch_types=[
            pltpu.VMEM((1568,), jnp.float32),
            pltpu.VMEM((1568,), jnp.float32),
            pltpu.VMEM((1568,), jnp.float32),
            pltpu.VMEM((128,), jnp.int32),
            pltpu.VMEM((128,), jnp.int32),
            pltpu.VMEM((128,), jnp.int32),
            pltpu.VMEM((128,), jnp.float32),
            pltpu.VMEM((128,), jnp.float32),
            pltpu.VMEM((128,), jnp.float32),
            pltpu.VMEM((128,), jnp.float32),
            pltpu.VMEM((16,), jnp.int32),
            pltpu.SemaphoreType.DMA,
            pltpu.VMEM_SHARED((N,), jnp.float32),
        ],
    )(_k2_body)
    return f(deg2, srcb, dstb, wb, nbat)


def _make_prop(F):
    FC = F // 16

    def body(xw, srcb, dstb, wnb, nbat, S,
             sidx, didx, wv, rows, sidx2, didx2, wv2, rows2, nbv, zb,
             gsem, ssem, ssem2, isem, S_sp):
        c = lax.axis_index("c")
        s = lax.axis_index("s")
        ii = lax.iota(jnp.int32, 16)
        zf = jnp.zeros((16,), jnp.float32)

        for r in range(49):
            for j in range(FC):
                zb[r, pl.ds(16 * j, 16)] = zf

        for phase in range(4):
            b = 4 * c + phase
            plsc.subcore_barrier()
            for i in range(8):
                pltpu.sync_copy(zb, S_sp.at[pl.ds((s * 8 + i) * 49, 49)])
            plsc.subcore_barrier()

            for t_off in range(2):
                t = 2 * s + t_off
                pltpu.sync_copy(nbat.at[t], nbv)
                nb_b = _extract_lane(nbv[...], b, ii)

                sx = [sidx, sidx2]
                dx = [didx, didx2]
                wx = [wv, wv2]
                rx = [rows, rows2]
                ss = [ssem, ssem2]

                def load_idx(k, p, t=t, b=b):
                    o = pl.multiple_of(k * 128, 128)
                    h1 = pltpu.async_copy(srcb.at[t, b, pl.ds(o, 128)],
                                          sx[p], isem)
                    h2 = pltpu.async_copy(dstb.at[t, b, pl.ds(o, 128)],
                                          dx[p], isem)
                    h3 = pltpu.async_copy(wnb.at[t, b, pl.ds(o, 128)],
                                          wx[p], isem)
                    h1.wait()
                    h2.wait()
                    h3.wait()

                def compute(p):
                    def e_outer(j, _2):
                        for i_ in range(8):
                            e = j * 8 + i_
                            wspl = plsc.load_gather(
                                wx[p], [jnp.full((16,), e, jnp.int32)])
                            for fc in range(FC):
                                v = rx[p][e, pl.ds(16 * fc, 16)].reshape(
                                    (16,)) * wspl
                                rx[p][e, pl.ds(16 * fc, 16)] = v
                        return 0

                    lax.fori_loop(0, 16, e_outer, 0)

                def drain_scatter(p):
                    pltpu.make_async_copy(rx[p], S_sp.at[dx[p]],
                                          ss[p]).wait()

                @pl.when(nb_b > 0)
                def _(t=t, b=b):
                    load_idx(0, 0)
                    pltpu.async_copy(xw.at[sx[0]], rx[0], gsem)

                def batch_body(k, _, t=t, b=b):
                    for p in range(2):
                        @pl.when((k & 1) == p)
                        def _(p=p, t=t, b=b):
                            q = 1 - p
                            pltpu.make_async_copy(xw.at[sx[p]], rx[p],
                                                  gsem).wait()

                            @pl.when(k + 1 < nb_b)
                            def _(p=p, q=q, t=t, b=b):
                                @pl.when(k >= 1)
                                def _(q=q):
                                    drain_scatter(q)
                                load_idx(k + 1, q)
                                pltpu.async_copy(xw.at[sx[q]], rx[q], gsem)

                            compute(p)
                            pltpu.async_copy(rx[p], S_sp.at[dx[p]], ss[p],
                                             add=True)
                    return 0

                lax.fori_loop(0, nb_b, batch_body, 0)

                for p in range(2):
                    @pl.when((nb_b >= 2) & (jnp.bitwise_and(nb_b, 1) == p))
                    def _(p=p):
                        drain_scatter(p)

                    @pl.when((nb_b >= 1)
                             & (jnp.bitwise_and(nb_b - 1, 1) == p))
                    def _(p=p):
                        drain_scatter(p)

            plsc.subcore_barrier()
            r0 = 392 * s

            @pl.when(s < 15)
            def _(b=b, r0=r0):
                pltpu.sync_copy(S_sp.at[pl.ds(r0, 392)],
                                S.at[pl.ds(b * BW + r0, 392)])

            @pl.when(s == 15)
            def _(b=b):
                pltpu.sync_copy(S_sp.at[pl.ds(5880, 370)],
                                S.at[pl.ds(b * BW + 5880, 370)])

    mesh = plsc.VectorSubcoreMesh(core_axis_name="c", subcore_axis_name="s")

    def run(xw, srcb, dstb, wnb, nbat):
        f = functools.partial(
            pl.kernel,
            mesh=mesh,
            compiler_params=pltpu.CompilerParams(
                needs_layout_passes=False, use_tc_tiling_on_sc=False),
            out_type=(jax.ShapeDtypeStruct((N, F), jnp.float32),),
            scratch_types=[
                pltpu.VMEM((128,), jnp.int32),
                pltpu.VMEM((128,), jnp.int32),
                pltpu.VMEM((128,), jnp.float32),
                pltpu.VMEM((128, F), jnp.float32),
                pltpu.VMEM((128,), jnp.int32),
                pltpu.VMEM((128,), jnp.int32),
                pltpu.VMEM((128,), jnp.float32),
                pltpu.VMEM((128, F), jnp.float32),
                pltpu.VMEM((16,), jnp.int32),
                pltpu.VMEM((49, F), jnp.float32),
                pltpu.SemaphoreType.DMA,
                pltpu.SemaphoreType.DMA,
                pltpu.SemaphoreType.DMA,
                pltpu.SemaphoreType.DMA,
                pltpu.VMEM_SHARED((BP, F), jnp.float32),
            ],
        )(body)
        res = f(xw, srcb, dstb, wnb, nbat)
        return res[0] if isinstance(res, (tuple, list)) else res

    return run


def _pool_body(S3, ac, batch, psum2, pcnt2,
               tb, bidx, tb48, bidx48, ones112, ones48, acv, zb, zc,
               gsem, psum_sp, pcnt_sp):
    c = lax.axis_index("c")
    s = lax.axis_index("s")
    wid = s * 2 + c
    zf = jnp.zeros((16,), jnp.float32)
    onef = jnp.full((16,), 1.0, jnp.float32)

    for r in range(64):
        for j in range(4):
            zb[r, pl.ds(16 * j, 16)] = zf
    for i in range(4):
        zc[pl.ds(16 * i, 16)] = zf
    for i in range(7):
        ones112[pl.ds(16 * i, 16)] = onef
    for i in range(3):
        ones48[pl.ds(16 * i, 16)] = onef

    pltpu.sync_copy(ac, acv)
    a_l = [acv[0, pl.ds(16 * j, 16)].reshape((16,)) for j in range(4)]
    c_l = [acv[1, pl.ds(16 * j, 16)].reshape((16,)) for j in range(4)]

    pltpu.sync_copy(zb, psum_sp.at[pl.ds(s * 64, 64)])
    pltpu.sync_copy(zc, pcnt_sp.at[pl.ds(s * 64, 64)])
    plsc.subcore_barrier()

    def do_chunk(buf, bx, ones, nrows, r0):
        pltpu.sync_copy(S3.at[pl.ds(r0, nrows)], buf)
        pltpu.sync_copy(batch.at[pl.ds(r0, nrows)], bx)

        def row_body(r8, _):
            for i_ in range(8):
                r = r8 * 8 + i_
                for j in range(4):
                    v = buf[r, pl.ds(16 * j, 16)].reshape((16,))
                    z = jnp.maximum(v * a_l[j] + c_l[j], 0.0)
                    buf[r, pl.ds(16 * j, 16)] = z
            return 0

        lax.fori_loop(0, nrows // 8, row_body, 0)
        pltpu.async_copy(buf, psum_sp.at[bx], gsem, add=True).wait()
        pltpu.async_copy(ones, pcnt_sp.at[bx], gsem, add=True).wait()

    @pl.when(wid < 31)
    def _():
        def chunk_loop(k, _):
            r0 = pl.multiple_of(wid * 1568 + k * 112, 8)
            do_chunk(tb, bidx, ones112, 112, r0)
            return 0
        lax.fori_loop(0, 14, chunk_loop, 0)

    @pl.when(wid == 31)
    def _():
        def chunk_loop(k, _):
            r0 = pl.multiple_of(48608 + k * 112, 8)
            do_chunk(tb, bidx, ones112, 112, r0)
            return 0
        lax.fori_loop(0, 12, chunk_loop, 0)
        do_chunk(tb48, bidx48, ones48, 48, 49952)

    plsc.subcore_barrier()
    pltpu.sync_copy(psum_sp.at[pl.ds(s * 64, 64)],
                    psum2.at[c, pl.ds(s * 64, 64)])
    pltpu.sync_copy(pcnt_sp.at[pl.ds(s * 64, 64)],
                    pcnt2.at[c, pl.ds(s * 64, 64)])


def _pool(S3, ac, batch):
    mesh = plsc.VectorSubcoreMesh(core_axis_name="c", subcore_axis_name="s")
    f = functools.partial(
        pl.kernel,
        mesh=mesh,
        compiler_params=pltpu.CompilerParams(
            needs_layout_passes=False, use_tc_tiling_on_sc=False),
        out_type=(
            jax.ShapeDtypeStruct((2, NUM_GRAPHS, 64), jnp.float32),
            jax.ShapeDtypeStruct((2, NUM_GRAPHS), jnp.float32),
        ),
        scratch_types=[
            pltpu.VMEM((112, 64), jnp.float32),
            pltpu.VMEM((112,), jnp.int32),
            pltpu.VMEM((48, 64), jnp.float32),
            pltpu.VMEM((48,), jnp.int32),
            pltpu.VMEM((112,), jnp.float32),
            pltpu.VMEM((48,), jnp.float32),
            pltpu.VMEM((2, 64), jnp.float32),
            pltpu.VMEM((64, 64), jnp.float32),
            pltpu.VMEM((64,), jnp.float32),
            pltpu.SemaphoreType.DMA,
            pltpu.VMEM_SHARED((NUM_GRAPHS, 64), jnp.float32),
            pltpu.VMEM_SHARED((NUM_GRAPHS,), jnp.float32),
        ],
    )(_pool_body)
    return f(S3, ac, batch)


def _tc1_kernel(x_ref, wc_ref, cb_ref, w1_ref, out_ref):
    h = jnp.maximum(
        jnp.dot(x_ref[...], wc_ref[...],
                preferred_element_type=jnp.float32) + cb_ref[...], 0.0)
    out_ref[...] = jnp.dot(h, w1_ref[...], preferred_element_type=jnp.float32)


def _tc1(x, wc, cb, w1):
    return pl.pallas_call(
        _tc1_kernel,
        grid=(NBLK,),
        in_specs=[
            pl.BlockSpec((BM, 200), lambda i: (i, 0)),
            pl.BlockSpec((200, 192), lambda i: (0, 0)),
            pl.BlockSpec((1, 192), lambda i: (0, 0)),
            pl.BlockSpec((192, 128), lambda i: (0, 0)),
        ],
        out_specs=pl.BlockSpec((BM, 128), lambda i: (i, 0)),
        out_shape=jax.ShapeDtypeStruct((N, 128), jnp.float32),
    )(x, wc, cb, w1)


def _make_tcb(F):
    def kern(s_ref, b_ref, g_ref, be_ref, out_ref, acc_ref):
        i = pl.program_id(0)

        @pl.when(i == 0)
        def _():
            acc_ref[...] = jnp.zeros_like(acc_ref)

        t = s_ref[...] + b_ref[...]
        acc_ref[0:1, :] += jnp.sum(t, axis=0, keepdims=True)
        acc_ref[1:2, :] += jnp.sum(t * t, axis=0, keepdims=True)

        @pl.when(i == NBLK - 1)
        def _():
            mu = acc_ref[0:1, :] / N
            var = acc_ref[1:2, :] / N - mu * mu
            rs = lax.rsqrt(var + 1e-5)
            a = rs * g_ref[...]
            out_ref[0:1, :] = a
            out_ref[1:2, :] = (b_ref[...] - mu) * a + be_ref[...]

    def run(S, b, g, be):
        return pl.pallas_call(
            kern,
            grid=(NBLK,),
            in_specs=[
                pl.BlockSpec((BM, F), lambda i: (i, 0)),
                pl.BlockSpec((1, F), lambda i: (0, 0)),
                pl.BlockSpec((1, F), lambda i: (0, 0)),
                pl.BlockSpec((1, F), lambda i: (0, 0)),
            ],
            out_specs=pl.BlockSpec((2, F), lambda i: (0, 0)),
            out_shape=jax.ShapeDtypeStruct((2, F), jnp.float32),
            scratch_shapes=[pltpu.VMEM((2, F), jnp.float32)],
        )(S, b.reshape(1, F), g.reshape(1, F), be.reshape(1, F))

    return run


def _make_tc23(F, FO):
    def kern(s_ref, ac_ref, w_ref, out_ref):
        z = jnp.maximum(s_ref[...] * ac_ref[0:1, :] + ac_ref[1:2, :], 0.0)
        out_ref[...] = jnp.dot(z, w_ref[...],
                               preferred_element_type=jnp.float32)

    def run(S, ac, w):
        return pl.pallas_call(
            kern,
            grid=(NBLK,),
            in_specs=[
                pl.BlockSpec((BM, F), lambda i: (i, 0)),
                pl.BlockSpec((2, F), lambda i: (0, 0)),
                pl.BlockSpec((F, FO), lambda i: (0, 0)),
            ],
            out_specs=pl.BlockSpec((BM, FO), lambda i: (i, 0)),
            out_shape=jax.ShapeDtypeStruct((N, FO), jnp.float32),
        )(S, ac, w)

    return run


def _tcf_kernel(ps_ref, pc_ref, fw_ref, fb_ref, out_ref):
    ps = ps_ref[0] + ps_ref[1]
    cnt = pc_ref[0] + pc_ref[1]
    pooled = ps / jnp.maximum(cnt, 1.0)[:, None]
    logits = jnp.dot(pooled, fw_ref[...],
                     preferred_element_type=jnp.float32) + fb_ref[...]
    col = lax.broadcasted_iota(jnp.int32, logits.shape, 1)
    logits = jnp.where(col < 4, logits, -1e30)
    m = jnp.max(logits, axis=1, keepdims=True)
    sh = logits - m
    lse = jnp.log(jnp.sum(jnp.exp(sh), axis=1, keepdims=True))
    out_ref[...] = sh - lse


def _tcf(psum2, pcnt2, fw, fb):
    return pl.pallas_call(
        _tcf_kernel,
        out_shape=jax.ShapeDtypeStruct((NUM_GRAPHS, 128), jnp.float32),
    )(psum2, pcnt2, fw, fb)


_DBG = "full"


def kernel(x, edge_index, edge_attr, batch, conv_w, conv_b, W1, b1, W2, b2, W3, b3, g1, be1, g2, be2, g3, be3, fc_w, fc_b):
    loop = jnp.arange(N, dtype=jnp.int32)
    pad = ROWS * 128 - EA
    srcA = jnp.concatenate(
        [edge_index[0], loop, jnp.zeros((pad,), jnp.int32)]).reshape(ROWS, 128)
    dstA = jnp.concatenate(
        [edge_index[1], loop, jnp.zeros((pad,), jnp.int32)]).reshape(ROWS, 128)
    wA = jnp.concatenate(
        [edge_attr, jnp.ones((N,), jnp.float32),
         jnp.zeros((pad,), jnp.float32)]).reshape(ROWS, 128)

    deg2, srcb, dstb, wb, nbat = _k1(srcA, dstA, wA)
    wnb = _k2(deg2, srcb, dstb, wb, nbat)
    if isinstance(wnb, (tuple, list)):
        wnb = wnb[0]

    wct = conv_w[:, 0, :].T
    wc = jnp.zeros((200, 32, 6), jnp.float32)
    for r in range(6):
        wc = wc.at[20 * r:20 * r + 100, :, r].set(wct)
    wc = wc.reshape(200, 192)
    cb = jnp.repeat(conv_b, 6).reshape(1, 192)

    p128 = _make_prop(128)
    p64 = _make_prop(64)
    tcb128 = _make_tcb(128)
    tcb64 = _make_tcb(64)

    def prop_jnp(q):
        cnt = nbat[:, :NB] * 128
        ar = jnp.arange(CAP)[None, None, :]
        valid = ar < cnt[:, :, None]
        w_eff = jnp.where(valid, wnb, 0.0).reshape(-1)
        src_eff = jnp.where(valid, srcb, 0).reshape(-1)
        dg = jnp.clip(dstb, 0, BW - 1) + (
            jnp.arange(NB, dtype=jnp.int32)[None, :, None] * BW)
        dst_eff = jnp.where(valid, dg, 0).reshape(-1)
        msg = q[src_eff] * w_eff[:, None]
        return jnp.zeros_like(q).at[dst_eff].add(msg)

    def bn_jnp(t, g, be):
        mu = jnp.mean(t, axis=0)
        va = jnp.var(t, axis=0)
        return (t - mu) * lax.rsqrt(va + 1e-5) * g + be

    if _DBG in ("k12", "prop"):
        prop = prop_jnp if _DBG == "k12" else (
            lambda q, F=None: None)
        xw1j = jnp.maximum(x @ wc + cb, 0.0) @ W1
        if _DBG == "k12":
            S1j = prop_jnp(xw1j)
        else:
            S1j = p128(xw1j, srcb, dstb, wnb, nbat)
        h = jnp.maximum(bn_jnp(S1j + b1, g1, be1), 0.0)
        xw2j = h @ W2
        S2j = prop_jnp(xw2j) if _DBG == "k12" else p128(
            xw2j, srcb, dstb, wnb, nbat)
        h = jnp.maximum(bn_jnp(S2j + b2, g2, be2), 0.0)
        xw3j = h @ W3
        S3j = prop_jnp(xw3j) if _DBG == "k12" else p64(
            xw3j, srcb, dstb, wnb, nbat)
        h = jnp.maximum(bn_jnp(S3j + b3, g3, be3), 0.0)
        sums = jax.ops.segment_sum(h, batch, num_segments=NUM_GRAPHS)
        cntg = jax.ops.segment_sum(jnp.ones((N,), jnp.float32), batch,
                                   num_segments=NUM_GRAPHS)
        pooled = sums / jnp.maximum(cntg, 1.0)[:, None]
        logits = pooled @ fc_w + fc_b
        return jax.nn.log_softmax(logits, axis=1)

    xw1 = _tc1(x, wc, cb, W1)
    S1 = prop_jnp(xw1) if _DBG == "tc" else p128(xw1, srcb, dstb, wnb, nbat)
    ac1 = tcb128(S1, b1, g1, be1)
    xw2 = _make_tc23(128, 128)(S1, ac1, W2)
    S2 = prop_jnp(xw2) if _DBG == "tc" else p128(xw2, srcb, dstb, wnb, nbat)
    ac2 = tcb128(S2, b2, g2, be2)
    xw3 = _make_tc23(128, 64)(S2, ac2, W3)
    S3 = prop_jnp(xw3) if _DBG == "tc" else p64(xw3, srcb, dstb, wnb, nbat)
    ac3 = tcb64(S3, b3, g3, be3)
    psum2, pcnt2 = _pool(S3, ac3, batch)

    fw = jnp.zeros((64, 128), jnp.float32).at[:, :4].set(fc_w)
    fb = jnp.zeros((1, 128), jnp.float32).at[0, :4].set(fc_b)
    out = _tcf(psum2, pcnt2, fw, fb)
    return out[:, :4]


# speedup vs baseline: 12.2845x; 1.0944x over previous
import functools

import jax
import jax.numpy as jnp
from jax import lax
from jax.experimental import pallas as pl
from jax.experimental.pallas import tpu as pltpu
from jax.experimental.pallas import tpu_sc as plsc

N = 50000
E0 = 800000
EA = E0 + N
NUM_GRAPHS = 1024
NB = 8
BW = 6250
BP = 6272
ROWS = 6648
NCHUNK = ROWS // 8
NWORK = 32
CAP = 27648
STG = 2320
BM = 2000
NBLK = N // BM


def _i32(x):
    return x.astype(jnp.int32)


def _extract_lane(vec, lane_const, ii):
    sel = jnp.where(ii == lane_const, vec, jnp.zeros_like(vec))
    return jnp.sum(sel)


def _k1_body(src_hbm, dst_hbm, w_hbm, deg2, srcb, dstb, wb, nbat,
             in_src, in_dst, in_w, st_src, st_dst, st_w, zv, nbv,
             dsem, deg_sp):
    c = lax.axis_index("c")
    s = lax.axis_index("s")
    wid = s * 2 + c
    ii = lax.iota(jnp.int32, 16)
    zf = jnp.zeros((16,), jnp.float32)

    for i in range(200):
        zv[pl.ds(16 * i, 16)] = zf

    @pl.when(s < 15)
    def _():
        pltpu.sync_copy(zv.at[pl.ds(0, 3200)],
                        deg_sp.at[pl.ds(pl.multiple_of(s * 3200, 128), 3200)])

    @pl.when(s == 15)
    def _():
        pltpu.sync_copy(zv.at[pl.ds(0, 2000)], deg_sp.at[pl.ds(48000, 2000)])

    plsc.subcore_barrier()

    nck = (NCHUNK - wid + NWORK - 1) // NWORK

    def chunk_body(k, carry):
        off = list(carry[0:NB])
        wr = list(carry[NB:2 * NB])
        r0 = pl.multiple_of((wid + NWORK * k) * 8, 8)
        pltpu.sync_copy(src_hbm.at[pl.ds(r0, 8)], in_src)
        pltpu.sync_copy(dst_hbm.at[pl.ds(r0, 8)], in_dst)
        pltpu.sync_copy(w_hbm.at[pl.ds(r0, 8)], in_w)

        handles = []
        for r in range(8):
            handles.append(pltpu.async_copy(
                in_w.at[r], deg_sp.at[in_dst.at[r]], dsem, add=True))
        for h in handles:
            h.wait()

        def group_body(g, gc):
            goff = list(gc)
            r = g // 8
            kk = g - 8 * r
            d = in_dst[r, pl.ds(kk * 16, 16)].reshape((16,))
            sv = in_src[r, pl.ds(kk * 16, 16)].reshape((16,))
            wv = in_w[r, pl.ds(kk * 16, 16)].reshape((16,))
            bid = sum(_i32(d >= kk * BW) for kk in range(1, NB))
            dl = d - bid * BW
            for b in range(NB):
                m = bid == b
                plsc.store_compressed(
                    st_src.at[pl.ds(b * STG + goff[b], 16)], sv, mask=m)
                plsc.store_compressed(
                    st_dst.at[pl.ds(b * STG + goff[b], 16)], dl, mask=m)
                plsc.store_compressed(
                    st_w.at[pl.ds(b * STG + goff[b], 16)], wv, mask=m)
                goff[b] = goff[b] + jnp.sum(_i32(m))
            return tuple(goff)

        off = list(lax.fori_loop(0, 64, group_body, tuple(off)))

        for b in range(NB):
            do = off[b] >= 1024

            @pl.when(do)
            def _(b=b, wrb=wr[b]):
                o = pl.multiple_of(wrb * 1024, 128)
                pltpu.sync_copy(st_src.at[pl.ds(b * STG, 1024)],
                                srcb.at[wid, b, pl.ds(o, 1024)])
                pltpu.sync_copy(st_dst.at[pl.ds(b * STG, 1024)],
                                dstb.at[wid, b, pl.ds(o, 1024)])
                pltpu.sync_copy(st_w.at[pl.ds(b * STG, 1024)],
                                wb.at[wid, b, pl.ds(o, 1024)])
                for i in range(64):
                    o_hi = b * STG + 1024 + 16 * i
                    o_lo = b * STG + 16 * i
                    st_src[pl.ds(o_lo, 16)] = st_src[pl.ds(o_hi, 16)]
                    st_dst[pl.ds(o_lo, 16)] = st_dst[pl.ds(o_hi, 16)]
                    st_w[pl.ds(o_lo, 16)] = st_w[pl.ds(o_hi, 16)]

            di = _i32(do)
            wr[b] = wr[b] + di
            off[b] = off[b] - 1024 * di
        return tuple(off) + tuple(wr)

    carry = lax.fori_loop(0, nck, chunk_body, (0,) * (2 * NB))
    off = list(carry[0:NB])
    wr = list(carry[NB:2 * NB])

    nbvec = jnp.zeros((16,), jnp.int32)
    for b in range(NB):
        for i in range(8):
            st_src[pl.ds(b * STG + off[b] + 16 * i, 16)] = ii
            st_dst[pl.ds(b * STG + off[b] + 16 * i, 16)] = ii
            st_w[pl.ds(b * STG + off[b] + 16 * i, 16)] = zf
        nblk = (off[b] + 127) // 128
        for i in range(8):
            @pl.when(i < nblk)
            def _(b=b, i=i, wrb=wr[b]):
                o = pl.multiple_of(wrb * 1024 + 128 * i, 128)
                pltpu.sync_copy(st_src.at[pl.ds(b * STG + 128 * i, 128)],
                                srcb.at[wid, b, pl.ds(o, 128)])
                pltpu.sync_copy(st_dst.at[pl.ds(b * STG + 128 * i, 128)],
                                dstb.at[wid, b, pl.ds(o, 128)])
                pltpu.sync_copy(st_w.at[pl.ds(b * STG + 128 * i, 128)],
                                wb.at[wid, b, pl.ds(o, 128)])
        tot = wr[b] * 8 + nblk
        nbvec = jnp.where(ii == b, jnp.full((16,), tot, jnp.int32), nbvec)
    nbv[...] = nbvec
    pltpu.sync_copy(nbv, nbat.at[wid])

    plsc.subcore_barrier()

    @pl.when(s < 15)
    def _():
        o = pl.multiple_of(s * 3200, 128)
        pltpu.sync_copy(deg_sp.at[pl.ds(o, 3200)], deg2.at[c, pl.ds(o, 3200)])

    @pl.when(s == 15)
    def _():
        pltpu.sync_copy(deg_sp.at[pl.ds(48000, 2000)],
                        deg2.at[c, pl.ds(48000, 2000)])


def _k1(src2d, dst2d, w2d):
    mesh = plsc.VectorSubcoreMesh(core_axis_name="c", subcore_axis_name="s")
    f = functools.partial(
        pl.kernel,
        mesh=mesh,
        compiler_params=pltpu.CompilerParams(
            needs_layout_passes=False, use_tc_tiling_on_sc=False),
        out_type=(
            jax.ShapeDtypeStruct((2, N), jnp.float32),
            jax.ShapeDtypeStruct((NWORK, NB, CAP), jnp.int32),
            jax.ShapeDtypeStruct((NWORK, NB, CAP), jnp.int32),
            jax.ShapeDtypeStruct((NWORK, NB, CAP), jnp.float32),
            jax.ShapeDtypeStruct((NWORK, 16), jnp.int32),
        ),
        scratch_types=[
            pltpu.VMEM((8, 128), jnp.int32),
            pltpu.VMEM((8, 128), jnp.int32),
            pltpu.VMEM((8, 128), jnp.float32),
            pltpu.VMEM((NB * STG,), jnp.int32),
            pltpu.VMEM((NB * STG,), jnp.int32),
            pltpu.VMEM((NB * STG,), jnp.float32),
            pltpu.VMEM((3200,), jnp.float32),
            pltpu.VMEM((16,), jnp.int32),
            pltpu.SemaphoreType.DMA,
            pltpu.VMEM_SHARED((N,), jnp.float32),
        ],
    )(_k1_body)
    return f(src2d, dst2d, w2d)


def _k2_body(deg2, srcb, dstb, wb, nbat, wnb,
             da, db, dv, sidx, didx, dgi, wv, dsv, ddv, wn,
             sidx2, didx2, dgi2, wv2, dsv2, ddv2, wn2, nbv,
             gsem, gsem2, isem, osem, osem2, dinv_sp):
    c = lax.axis_index("c")
    s = lax.axis_index("s")
    wid = s * 2 + c
    ii = lax.iota(jnp.int32, 16)

    def newton(nv):
        half = jnp.full((16,), 0.5, jnp.float32)
        threeh = jnp.full((16,), 1.5, jnp.float32)
        magic = jnp.full((16,), 0x5f3759df, jnp.int32)
        for i in range(nv):
            x = da[pl.ds(16 * i, 16)] + db[pl.ds(16 * i, 16)]
            bits = plsc.bitcast(x, jnp.int32)
            y = plsc.bitcast(magic - lax.shift_right_logical(bits, jnp.full((16,), 1, jnp.int32)),
                             jnp.float32)
            for _ in range(3):
                y = y * (threeh - half * x * y * y)
            dv[pl.ds(16 * i, 16)] = y

    for half in range(2):
        sl = 2 * s + half

        @pl.when(sl < 31)
        def _(sl=sl):
            r0 = pl.multiple_of(sl * 1568, 8)
            pltpu.sync_copy(deg2.at[0, pl.ds(r0, 1568)], da)
            pltpu.sync_copy(deg2.at[1, pl.ds(r0, 1568)], db)

        @pl.when(sl == 31)
        def _():
            pltpu.sync_copy(deg2.at[0, pl.ds(48608, 1392)],
                            da.at[pl.ds(0, 1392)])
            pltpu.sync_copy(deg2.at[1, pl.ds(48608, 1392)],
                            db.at[pl.ds(0, 1392)])

        newton(98)

        @pl.when(sl < 31)
        def _(sl=sl):
            r0 = pl.multiple_of(sl * 1568, 8)
            pltpu.sync_copy(dv, dinv_sp.at[pl.ds(r0, 1568)])

        @pl.when(sl == 31)
        def _():
            pltpu.sync_copy(dv.at[pl.ds(0, 1392)],
                            dinv_sp.at[pl.ds(48608, 1392)])

    plsc.subcore_barrier()

    pltpu.sync_copy(nbat.at[wid], nbv)
    nbvec = nbv[...]
    sx = [sidx, sidx2]
    dx = [didx, didx2]
    gx = [dgi, dgi2]
    wx = [wv, wv2]
    dsx = [dsv, dsv2]
    ddx = [ddv, ddv2]
    wnx = [wn, wn2]
    gs = [gsem, gsem2]
    os_ = [osem, osem2]
    for b in range(NB):
        nb_b = _extract_lane(nbvec, b, ii)
        base = jnp.full((16,), b * BW, jnp.int32)

        def stage_in(k, p, b=b, base=base):
            o = pl.multiple_of(k * 128, 128)
            h1 = pltpu.async_copy(srcb.at[wid, b, pl.ds(o, 128)], sx[p], isem)
            h2 = pltpu.async_copy(dstb.at[wid, b, pl.ds(o, 128)], dx[p], isem)
            h3 = pltpu.async_copy(wb.at[wid, b, pl.ds(o, 128)], wx[p], isem)
            h1.wait()
            h2.wait()
            h3.wait()
            for j in range(8):
                gx[p][pl.ds(16 * j, 16)] = dx[p][pl.ds(16 * j, 16)] + base
            pltpu.async_copy(dinv_sp.at[sx[p]], dsx[p], gs[p])
            pltpu.async_copy(dinv_sp.at[gx[p]], ddx[p], gs[p])

        def finish(k, p, b=b):
            o = pl.multiple_of(k * 128, 128)
            pltpu.make_async_copy(dinv_sp.at[sx[p]], dsx[p], gs[p]).wait()
            pltpu.make_async_copy(dinv_sp.at[gx[p]], ddx[p], gs[p]).wait()
            for j in range(8):
                wnx[p][pl.ds(16 * j, 16)] = (wx[p][pl.ds(16 * j, 16)]
                                             * dsx[p][pl.ds(16 * j, 16)]
                                             * ddx[p][pl.ds(16 * j, 16)])
            pltpu.async_copy(wnx[p], wnb.at[wid, b, pl.ds(o, 128)], os_[p])

        def drain_out(p, b=b):
            pltpu.make_async_copy(
                wnx[p], wnb.at[wid, b, pl.ds(0, 128)], os_[p]).wait()

        @pl.when(nb_b > 0)
        def _(b=b):
            stage_in(0, 0)

        def batch_body(k, _, b=b):
            for p in range(2):
                @pl.when((k & 1) == p)
                def _(p=p, b=b):
                    q = 1 - p

                    @pl.when(k + 1 < nb_b)
                    def _(p=p, q=q, b=b):
                        @pl.when(k >= 1)
                        def _(q=q):
                            drain_out(q)
                        stage_in(k + 1, q)
                    finish(k, p)
            return 0

        lax.fori_loop(0, nb_b, batch_body, 0)

        for p in range(2):
            @pl.when((nb_b >= 2) & (jnp.bitwise_and(nb_b, 1) == p))
            def _(p=p):
                drain_out(p)

            @pl.when((nb_b >= 1) & (jnp.bitwise_and(nb_b - 1, 1) == p))
            def _(p=p):
                drain_out(p)


def _k2(deg2, srcb, dstb, wb, nbat):
    mesh = plsc.VectorSubcoreMesh(core_axis_name="c", subcore_axis_name="s")
    f = functools.partial(
        pl.kernel,
        mesh=mesh,
        compiler_params=pltpu.CompilerParams(
            needs_layout_passes=False, use_tc_tiling_on_sc=False),
        out_type=(jax.ShapeDtypeStruct((NWORK, NB, CAP), jnp.float32),),
        scratch_types=[
            pltpu.VMEM((1568,), jnp.float32),
            pltpu.VMEM((1568,), jnp.float32),
            pltpu.VMEM((1568,), jnp.float32),
            pltpu.VMEM((128,), jnp.int32),
            pltpu.VMEM((128,), jnp.int32),
            pltpu.VMEM((128,), jnp.int32),
            pltpu.VMEM((128,), jnp.float32),
            pltpu.VMEM((128,), jnp.float32),
            pltpu.VMEM((128,), jnp.float32),
            pltpu.VMEM((128,), jnp.float32),
            pltpu.VMEM((128,), jnp.int32),
            pltpu.VMEM((128,), jnp.int32),
            pltpu.VMEM((128,), jnp.int32),
            pltpu.VMEM((128,), jnp.float32),
            pltpu.VMEM((128,), jnp.float32),
            pltpu.VMEM((128,), jnp.float32),
            pltpu.VMEM((128,), jnp.float32),
            pltpu.VMEM((16,), jnp.int32),
            pltpu.SemaphoreType.DMA,
            pltpu.SemaphoreType.DMA,
            pltpu.SemaphoreType.DMA,
            pltpu.SemaphoreType.DMA,
            pltpu.SemaphoreType.DMA,
            pltpu.VMEM_SHARED((N,), jnp.float32),
        ],
    )(_k2_body)
    return f(deg2, srcb, dstb, wb, nbat)


def _make_prop(F):
    FC = F // 16

    def body(xw, srcb, dstb, wnb, nbat, S,
             sidx, didx, wv, rows, sidx2, didx2, wv2, rows2, nbv, zb,
             gsem, ssem, ssem2, isem, S_sp):
        c = lax.axis_index("c")
        s = lax.axis_index("s")
        ii = lax.iota(jnp.int32, 16)
        zf = jnp.zeros((16,), jnp.float32)

        for r in range(49):
            for j in range(FC):
                zb[r, pl.ds(16 * j, 16)] = zf

        for phase in range(4):
            b = 4 * c + phase
            plsc.subcore_barrier()
            for i in range(8):
                pltpu.sync_copy(zb, S_sp.at[pl.ds((s * 8 + i) * 49, 49)])
            plsc.subcore_barrier()

            for t_off in range(2):
                t = 2 * s + t_off
                pltpu.sync_copy(nbat.at[t], nbv)
                nb_b = _extract_lane(nbv[...], b, ii)

                sx = [sidx, sidx2]
                dx = [didx, didx2]
                wx = [wv, wv2]
                rx = [rows, rows2]
                ss = [ssem, ssem2]

                def load_idx(k, p, t=t, b=b):
                    o = pl.multiple_of(k * 128, 128)
                    h1 = pltpu.async_copy(srcb.at[t, b, pl.ds(o, 128)],
                                          sx[p], isem)
                    h2 = pltpu.async_copy(dstb.at[t, b, pl.ds(o, 128)],
                                          dx[p], isem)
                    h3 = pltpu.async_copy(wnb.at[t, b, pl.ds(o, 128)],
                                          wx[p], isem)
                    h1.wait()
                    h2.wait()
                    h3.wait()

                def compute(p):
                    def e_outer(j, _2):
                        for i_ in range(8):
                            e = j * 8 + i_
                            wspl = plsc.load_gather(
                                wx[p], [jnp.full((16,), e, jnp.int32)])
                            for fc in range(FC):
                                v = rx[p][e, pl.ds(16 * fc, 16)].reshape(
                                    (16,)) * wspl
                                rx[p][e, pl.ds(16 * fc, 16)] = v
                        return 0

                    lax.fori_loop(0, 16, e_outer, 0)

                def drain_scatter(p):
                    pltpu.make_async_copy(rx[p], S_sp.at[dx[p]],
                                          ss[p]).wait()

                @pl.when(nb_b > 0)
                def _(t=t, b=b):
                    load_idx(0, 0)
                    pltpu.async_copy(xw.at[sx[0]], rx[0], gsem)

                def batch_body(k, _, t=t, b=b):
                    for p in range(2):
                        @pl.when((k & 1) == p)
                        def _(p=p, t=t, b=b):
                            q = 1 - p
                            pltpu.make_async_copy(xw.at[sx[p]], rx[p],
                                                  gsem).wait()

                            @pl.when(k + 1 < nb_b)
                            def _(p=p, q=q, t=t, b=b):
                                @pl.when(k >= 1)
                                def _(q=q):
                                    drain_scatter(q)
                                load_idx(k + 1, q)
                                pltpu.async_copy(xw.at[sx[q]], rx[q], gsem)

                            compute(p)
                            pltpu.async_copy(rx[p], S_sp.at[dx[p]], ss[p],
                                             add=True)
                    return 0

                lax.fori_loop(0, nb_b, batch_body, 0)

                for p in range(2):
                    @pl.when((nb_b >= 2) & (jnp.bitwise_and(nb_b, 1) == p))
                    def _(p=p):
                        drain_scatter(p)

                    @pl.when((nb_b >= 1)
                             & (jnp.bitwise_and(nb_b - 1, 1) == p))
                    def _(p=p):
                        drain_scatter(p)

            plsc.subcore_barrier()
            r0 = 392 * s

            @pl.when(s < 15)
            def _(b=b, r0=r0):
                pltpu.sync_copy(S_sp.at[pl.ds(r0, 392)],
                                S.at[pl.ds(b * BW + r0, 392)])

            @pl.when(s == 15)
            def _(b=b):
                pltpu.sync_copy(S_sp.at[pl.ds(5880, 370)],
                                S.at[pl.ds(b * BW + 5880, 370)])

    mesh = plsc.VectorSubcoreMesh(core_axis_name="c", subcore_axis_name="s")

    def run(xw, srcb, dstb, wnb, nbat):
        f = functools.partial(
            pl.kernel,
            mesh=mesh,
            compiler_params=pltpu.CompilerParams(
                needs_layout_passes=False, use_tc_tiling_on_sc=False),
            out_type=(jax.ShapeDtypeStruct((N, F), jnp.float32),),
            scratch_types=[
                pltpu.VMEM((128,), jnp.int32),
                pltpu.VMEM((128,), jnp.int32),
                pltpu.VMEM((128,), jnp.float32),
                pltpu.VMEM((128, F), jnp.float32),
                pltpu.VMEM((128,), jnp.int32),
                pltpu.VMEM((128,), jnp.int32),
                pltpu.VMEM((128,), jnp.float32),
                pltpu.VMEM((128, F), jnp.float32),
                pltpu.VMEM((16,), jnp.int32),
                pltpu.VMEM((49, F), jnp.float32),
                pltpu.SemaphoreType.DMA,
                pltpu.SemaphoreType.DMA,
                pltpu.SemaphoreType.DMA,
                pltpu.SemaphoreType.DMA,
                pltpu.VMEM_SHARED((BP, F), jnp.float32),
            ],
        )(body)
        res = f(xw, srcb, dstb, wnb, nbat)
        return res[0] if isinstance(res, (tuple, list)) else res

    return run


def _pool_body(S3, ac, batch, psum2, pcnt2,
               tb, bidx, tb48, bidx48, ones112, ones48, acv, zb, zc,
               gsem, psum_sp, pcnt_sp):
    c = lax.axis_index("c")
    s = lax.axis_index("s")
    wid = s * 2 + c
    zf = jnp.zeros((16,), jnp.float32)
    onef = jnp.full((16,), 1.0, jnp.float32)

    for r in range(64):
        for j in range(4):
            zb[r, pl.ds(16 * j, 16)] = zf
    for i in range(4):
        zc[pl.ds(16 * i, 16)] = zf
    for i in range(7):
        ones112[pl.ds(16 * i, 16)] = onef
    for i in range(3):
        ones48[pl.ds(16 * i, 16)] = onef

    pltpu.sync_copy(ac, acv)
    a_l = [acv[0, pl.ds(16 * j, 16)].reshape((16,)) for j in range(4)]
    c_l = [acv[1, pl.ds(16 * j, 16)].reshape((16,)) for j in range(4)]

    pltpu.sync_copy(zb, psum_sp.at[pl.ds(s * 64, 64)])
    pltpu.sync_copy(zc, pcnt_sp.at[pl.ds(s * 64, 64)])
    plsc.subcore_barrier()

    def do_chunk(buf, bx, ones, nrows, r0):
        pltpu.sync_copy(S3.at[pl.ds(r0, nrows)], buf)
        pltpu.sync_copy(batch.at[pl.ds(r0, nrows)], bx)

        def row_body(r8, _):
            for i_ in range(8):
                r = r8 * 8 + i_
                for j in range(4):
                    v = buf[r, pl.ds(16 * j, 16)].reshape((16,))
                    z = jnp.maximum(v * a_l[j] + c_l[j], 0.0)
                    buf[r, pl.ds(16 * j, 16)] = z
            return 0

        lax.fori_loop(0, nrows // 8, row_body, 0)
        pltpu.async_copy(buf, psum_sp.at[bx], gsem, add=True).wait()
        pltpu.async_copy(ones, pcnt_sp.at[bx], gsem, add=True).wait()

    @pl.when(wid < 31)
    def _():
        def chunk_loop(k, _):
            r0 = pl.multiple_of(wid * 1568 + k * 112, 8)
            do_chunk(tb, bidx, ones112, 112, r0)
            return 0
        lax.fori_loop(0, 14, chunk_loop, 0)

    @pl.when(wid == 31)
    def _():
        def chunk_loop(k, _):
            r0 = pl.multiple_of(48608 + k * 112, 8)
            do_chunk(tb, bidx, ones112, 112, r0)
            return 0
        lax.fori_loop(0, 12, chunk_loop, 0)
        do_chunk(tb48, bidx48, ones48, 48, 49952)

    plsc.subcore_barrier()
    pltpu.sync_copy(psum_sp.at[pl.ds(s * 64, 64)],
                    psum2.at[c, pl.ds(s * 64, 64)])
    pltpu.sync_copy(pcnt_sp.at[pl.ds(s * 64, 64)],
                    pcnt2.at[c, pl.ds(s * 64, 64)])


def _pool(S3, ac, batch):
    mesh = plsc.VectorSubcoreMesh(core_axis_name="c", subcore_axis_name="s")
    f = functools.partial(
        pl.kernel,
        mesh=mesh,
        compiler_params=pltpu.CompilerParams(
            needs_layout_passes=False, use_tc_tiling_on_sc=False),
        out_type=(
            jax.ShapeDtypeStruct((2, NUM_GRAPHS, 64), jnp.float32),
            jax.ShapeDtypeStruct((2, NUM_GRAPHS), jnp.float32),
        ),
        scratch_types=[
            pltpu.VMEM((112, 64), jnp.float32),
            pltpu.VMEM((112,), jnp.int32),
            pltpu.VMEM((48, 64), jnp.float32),
            pltpu.VMEM((48,), jnp.int32),
            pltpu.VMEM((112,), jnp.float32),
            pltpu.VMEM((48,), jnp.float32),
            pltpu.VMEM((2, 64), jnp.float32),
            pltpu.VMEM((64, 64), jnp.float32),
            pltpu.VMEM((64,), jnp.float32),
            pltpu.SemaphoreType.DMA,
            pltpu.VMEM_SHARED((NUM_GRAPHS, 64), jnp.float32),
            pltpu.VMEM_SHARED((NUM_GRAPHS,), jnp.float32),
        ],
    )(_pool_body)
    return f(S3, ac, batch)


def _tc1_kernel(x_ref, wc_ref, cb_ref, w1_ref, out_ref):
    h = jnp.maximum(
        jnp.dot(x_ref[...], wc_ref[...],
                preferred_element_type=jnp.float32) + cb_ref[...], 0.0)
    out_ref[...] = jnp.dot(h, w1_ref[...], preferred_element_type=jnp.float32)


def _tc1(x, wc, cb, w1):
    return pl.pallas_call(
        _tc1_kernel,
        grid=(NBLK,),
        in_specs=[
            pl.BlockSpec((BM, 200), lambda i: (i, 0)),
            pl.BlockSpec((200, 192), lambda i: (0, 0)),
            pl.BlockSpec((1, 192), lambda i: (0, 0)),
            pl.BlockSpec((192, 128), lambda i: (0, 0)),
        ],
        out_specs=pl.BlockSpec((BM, 128), lambda i: (i, 0)),
        out_shape=jax.ShapeDtypeStruct((N, 128), jnp.float32),
    )(x, wc, cb, w1)


def _make_tcb(F):
    def kern(s_ref, b_ref, g_ref, be_ref, out_ref, acc_ref):
        i = pl.program_id(0)

        @pl.when(i == 0)
        def _():
            acc_ref[...] = jnp.zeros_like(acc_ref)

        t = s_ref[...] + b_ref[...]
        acc_ref[0:1, :] += jnp.sum(t, axis=0, keepdims=True)
        acc_ref[1:2, :] += jnp.sum(t * t, axis=0, keepdims=True)

        @pl.when(i == NBLK - 1)
        def _():
            mu = acc_ref[0:1, :] / N
            var = acc_ref[1:2, :] / N - mu * mu
            rs = lax.rsqrt(var + 1e-5)
            a = rs * g_ref[...]
            out_ref[0:1, :] = a
            out_ref[1:2, :] = (b_ref[...] - mu) * a + be_ref[...]

    def run(S, b, g, be):
        return pl.pallas_call(
            kern,
            grid=(NBLK,),
            in_specs=[
                pl.BlockSpec((BM, F), lambda i: (i, 0)),
                pl.BlockSpec((1, F), lambda i: (0, 0)),
                pl.BlockSpec((1, F), lambda i: (0, 0)),
                pl.BlockSpec((1, F), lambda i: (0, 0)),
            ],
            out_specs=pl.BlockSpec((2, F), lambda i: (0, 0)),
            out_shape=jax.ShapeDtypeStruct((2, F), jnp.float32),
            scratch_shapes=[pltpu.VMEM((2, F), jnp.float32)],
        )(S, b.reshape(1, F), g.reshape(1, F), be.reshape(1, F))

    return run


def _make_tc23(F, FO):
    def kern(s_ref, ac_ref, w_ref, out_ref):
        z = jnp.maximum(s_ref[...] * ac_ref[0:1, :] + ac_ref[1:2, :], 0.0)
        out_ref[...] = jnp.dot(z, w_ref[...],
                               preferred_element_type=jnp.float32)

    def run(S, ac, w):
        return pl.pallas_call(
            kern,
            grid=(NBLK,),
            in_specs=[
                pl.BlockSpec((BM, F), lambda i: (i, 0)),
                pl.BlockSpec((2, F), lambda i: (0, 0)),
                pl.BlockSpec((F, FO), lambda i: (0, 0)),
            ],
            out_specs=pl.BlockSpec((BM, FO), lambda i: (i, 0)),
            out_shape=jax.ShapeDtypeStruct((N, FO), jnp.float32),
        )(S, ac, w)

    return run


def _tcf_kernel(ps_ref, pc_ref, fw_ref, fb_ref, out_ref):
    ps = ps_ref[0] + ps_ref[1]
    cnt = pc_ref[0] + pc_ref[1]
    pooled = ps / jnp.maximum(cnt, 1.0)[:, None]
    logits = jnp.dot(pooled, fw_ref[...],
                     preferred_element_type=jnp.float32) + fb_ref[...]
    col = lax.broadcasted_iota(jnp.int32, logits.shape, 1)
    logits = jnp.where(col < 4, logits, -1e30)
    m = jnp.max(logits, axis=1, keepdims=True)
    sh = logits - m
    lse = jnp.log(jnp.sum(jnp.exp(sh), axis=1, keepdims=True))
    out_ref[...] = sh - lse


def _tcf(psum2, pcnt2, fw, fb):
    return pl.pallas_call(
        _tcf_kernel,
        out_shape=jax.ShapeDtypeStruct((NUM_GRAPHS, 128), jnp.float32),
    )(psum2, pcnt2, fw, fb)


_DBG = "full"


def kernel(x, edge_index, edge_attr, batch, conv_w, conv_b, W1, b1, W2, b2, W3, b3, g1, be1, g2, be2, g3, be3, fc_w, fc_b):
    loop = jnp.arange(N, dtype=jnp.int32)
    pad = ROWS * 128 - EA
    srcA = jnp.concatenate(
        [edge_index[0], loop, jnp.zeros((pad,), jnp.int32)]).reshape(ROWS, 128)
    dstA = jnp.concatenate(
        [edge_index[1], loop, jnp.zeros((pad,), jnp.int32)]).reshape(ROWS, 128)
    wA = jnp.concatenate(
        [edge_attr, jnp.ones((N,), jnp.float32),
         jnp.zeros((pad,), jnp.float32)]).reshape(ROWS, 128)

    deg2, srcb, dstb, wb, nbat = _k1(srcA, dstA, wA)
    wnb = _k2(deg2, srcb, dstb, wb, nbat)
    if isinstance(wnb, (tuple, list)):
        wnb = wnb[0]

    wct = conv_w[:, 0, :].T
    wc = jnp.zeros((200, 32, 6), jnp.float32)
    for r in range(6):
        wc = wc.at[20 * r:20 * r + 100, :, r].set(wct)
    wc = wc.reshape(200, 192)
    cb = jnp.repeat(conv_b, 6).reshape(1, 192)

    p128 = _make_prop(128)
    p64 = _make_prop(64)
    tcb128 = _make_tcb(128)
    tcb64 = _make_tcb(64)

    def prop_jnp(q):
        cnt = nbat[:, :NB] * 128
        ar = jnp.arange(CAP)[None, None, :]
        valid = ar < cnt[:, :, None]
        w_eff = jnp.where(valid, wnb, 0.0).reshape(-1)
        src_eff = jnp.where(valid, srcb, 0).reshape(-1)
        dg = jnp.clip(dstb, 0, BW - 1) + (
            jnp.arange(NB, dtype=jnp.int32)[None, :, None] * BW)
        dst_eff = jnp.where(valid, dg, 0).reshape(-1)
        msg = q[src_eff] * w_eff[:, None]
        return jnp.zeros_like(q).at[dst_eff].add(msg)

    def bn_jnp(t, g, be):
        mu = jnp.mean(t, axis=0)
        va = jnp.var(t, axis=0)
        return (t - mu) * lax.rsqrt(va + 1e-5) * g + be

    if _DBG in ("k12", "prop"):
        prop = prop_jnp if _DBG == "k12" else (
            lambda q, F=None: None)
        xw1j = jnp.maximum(x @ wc + cb, 0.0) @ W1
        if _DBG == "k12":
            S1j = prop_jnp(xw1j)
        else:
            S1j = p128(xw1j, srcb, dstb, wnb, nbat)
        h = jnp.maximum(bn_jnp(S1j + b1, g1, be1), 0.0)
        xw2j = h @ W2
        S2j = prop_jnp(xw2j) if _DBG == "k12" else p128(
            xw2j, srcb, dstb, wnb, nbat)
        h = jnp.maximum(bn_jnp(S2j + b2, g2, be2), 0.0)
        xw3j = h @ W3
        S3j = prop_jnp(xw3j) if _DBG == "k12" else p64(
            xw3j, srcb, dstb, wnb, nbat)
        h = jnp.maximum(bn_jnp(S3j + b3, g3, be3), 0.0)
        sums = jax.ops.segment_sum(h, batch, num_segments=NUM_GRAPHS)
        cntg = jax.ops.segment_sum(jnp.ones((N,), jnp.float32), batch,
                                   num_segments=NUM_GRAPHS)
        pooled = sums / jnp.maximum(cntg, 1.0)[:, None]
        logits = pooled @ fc_w + fc_b
        return jax.nn.log_softmax(logits, axis=1)

    xw1 = _tc1(x, wc, cb, W1)
    S1 = prop_jnp(xw1) if _DBG == "tc" else p128(xw1, srcb, dstb, wnb, nbat)
    ac1 = tcb128(S1, b1, g1, be1)
    xw2 = _make_tc23(128, 128)(S1, ac1, W2)
    S2 = prop_jnp(xw2) if _DBG == "tc" else p128(xw2, srcb, dstb, wnb, nbat)
    ac2 = tcb128(S2, b2, g2, be2)
    xw3 = _make_tc23(128, 64)(S2, ac2, W3)
    S3 = prop_jnp(xw3) if _DBG == "tc" else p64(xw3, srcb, dstb, wnb, nbat)
    ac3 = tcb64(S3, b3, g3, be3)
    psum2, pcnt2 = _pool(S3, ac3, batch)

    fw = jnp.zeros((64, 128), jnp.float32).at[:, :4].set(fc_w)
    fb = jnp.zeros((1, 128), jnp.float32).at[0, :4].set(fc_b)
    out = _tcf(psum2, pcnt2, fw, fb)
    return out[:, :4]


# speedup vs baseline: 12.9568x; 1.0547x over previous
import functools

import jax
import jax.numpy as jnp
from jax import lax
from jax.experimental import pallas as pl
from jax.experimental.pallas import tpu as pltpu
from jax.experimental.pallas import tpu_sc as plsc

N = 50000
E0 = 800000
EA = E0 + N
NUM_GRAPHS = 1024
NB = 8
BW = 6250
BP = 6272
ROWS = 6648
NCHUNK = ROWS // 8
NWORK = 32
CAP = 27648
STG = 2320
BM = 2000
NBLK = N // BM


def _i32(x):
    return x.astype(jnp.int32)


def _extract_lane(vec, lane_const, ii):
    sel = jnp.where(ii == lane_const, vec, jnp.zeros_like(vec))
    return jnp.sum(sel)


def _k1_body(src_hbm, dst_hbm, w_hbm, deg2, srcb, dstb, wb, nbat,
             in_src, in_dst, in_w, st_src, st_dst, st_w, zv, nbv,
             dsem, deg_sp):
    c = lax.axis_index("c")
    s = lax.axis_index("s")
    wid = s * 2 + c
    ii = lax.iota(jnp.int32, 16)
    zf = jnp.zeros((16,), jnp.float32)

    for i in range(200):
        zv[pl.ds(16 * i, 16)] = zf

    @pl.when(s < 15)
    def _():
        pltpu.sync_copy(zv.at[pl.ds(0, 3200)],
                        deg_sp.at[pl.ds(pl.multiple_of(s * 3200, 128), 3200)])

    @pl.when(s == 15)
    def _():
        pltpu.sync_copy(zv.at[pl.ds(0, 2000)], deg_sp.at[pl.ds(48000, 2000)])

    plsc.subcore_barrier()

    nck = (NCHUNK - wid + NWORK - 1) // NWORK

    def chunk_body(k, carry):
        off = list(carry[0:NB])
        wr = list(carry[NB:2 * NB])
        r0 = pl.multiple_of((wid + NWORK * k) * 8, 8)
        pltpu.sync_copy(src_hbm.at[pl.ds(r0, 8)], in_src)
        pltpu.sync_copy(dst_hbm.at[pl.ds(r0, 8)], in_dst)
        pltpu.sync_copy(w_hbm.at[pl.ds(r0, 8)], in_w)

        handles = []
        for r in range(8):
            handles.append(pltpu.async_copy(
                in_w.at[r], deg_sp.at[in_dst.at[r]], dsem, add=True))
        for h in handles:
            h.wait()

        def group_body(g, gc):
            goff = list(gc)
            r = g // 8
            kk = g - 8 * r
            d = in_dst[r, pl.ds(kk * 16, 16)].reshape((16,))
            sv = in_src[r, pl.ds(kk * 16, 16)].reshape((16,))
            wv = in_w[r, pl.ds(kk * 16, 16)].reshape((16,))
            bid = sum(_i32(d >= kk * BW) for kk in range(1, NB))
            dl = d - bid * BW
            for b in range(NB):
                m = bid == b
                plsc.store_compressed(
                    st_src.at[pl.ds(b * STG + goff[b], 16)], sv, mask=m)
                plsc.store_compressed(
                    st_dst.at[pl.ds(b * STG + goff[b], 16)], dl, mask=m)
                plsc.store_compressed(
                    st_w.at[pl.ds(b * STG + goff[b], 16)], wv, mask=m)
                goff[b] = goff[b] + jnp.sum(_i32(m))
            return tuple(goff)

        off = list(lax.fori_loop(0, 64, group_body, tuple(off)))

        for b in range(NB):
            do = off[b] >= 1024

            @pl.when(do)
            def _(b=b, wrb=wr[b]):
                o = pl.multiple_of(wrb * 1024, 128)
                pltpu.sync_copy(st_src.at[pl.ds(b * STG, 1024)],
                                srcb.at[wid, b, pl.ds(o, 1024)])
                pltpu.sync_copy(st_dst.at[pl.ds(b * STG, 1024)],
                                dstb.at[wid, b, pl.ds(o, 1024)])
                pltpu.sync_copy(st_w.at[pl.ds(b * STG, 1024)],
                                wb.at[wid, b, pl.ds(o, 1024)])
                for i in range(64):
                    o_hi = b * STG + 1024 + 16 * i
                    o_lo = b * STG + 16 * i
                    st_src[pl.ds(o_lo, 16)] = st_src[pl.ds(o_hi, 16)]
                    st_dst[pl.ds(o_lo, 16)] = st_dst[pl.ds(o_hi, 16)]
                    st_w[pl.ds(o_lo, 16)] = st_w[pl.ds(o_hi, 16)]

            di = _i32(do)
            wr[b] = wr[b] + di
            off[b] = off[b] - 1024 * di
        return tuple(off) + tuple(wr)

    carry = lax.fori_loop(0, nck, chunk_body, (0,) * (2 * NB))
    off = list(carry[0:NB])
    wr = list(carry[NB:2 * NB])

    nbvec = jnp.zeros((16,), jnp.int32)
    for b in range(NB):
        for i in range(8):
            st_src[pl.ds(b * STG + off[b] + 16 * i, 16)] = ii
            st_dst[pl.ds(b * STG + off[b] + 16 * i, 16)] = ii
            st_w[pl.ds(b * STG + off[b] + 16 * i, 16)] = zf
        nblk = (off[b] + 127) // 128
        for i in range(8):
            @pl.when(i < nblk)
            def _(b=b, i=i, wrb=wr[b]):
                o = pl.multiple_of(wrb * 1024 + 128 * i, 128)
                pltpu.sync_copy(st_src.at[pl.ds(b * STG + 128 * i, 128)],
                                srcb.at[wid, b, pl.ds(o, 128)])
                pltpu.sync_copy(st_dst.at[pl.ds(b * STG + 128 * i, 128)],
                                dstb.at[wid, b, pl.ds(o, 128)])
                pltpu.sync_copy(st_w.at[pl.ds(b * STG + 128 * i, 128)],
                                wb.at[wid, b, pl.ds(o, 128)])
        tot = wr[b] * 8 + nblk
        odd = jnp.bitwise_and(tot, 1)

        @pl.when(odd == 1)
        def _(b=b, tot=tot):
            for i in range(8):
                st_src[pl.ds(b * STG + 16 * i, 16)] = ii
                st_dst[pl.ds(b * STG + 16 * i, 16)] = ii
                st_w[pl.ds(b * STG + 16 * i, 16)] = zf
            o = pl.multiple_of(tot * 128, 128)
            pltpu.sync_copy(st_src.at[pl.ds(b * STG, 128)],
                            srcb.at[wid, b, pl.ds(o, 128)])
            pltpu.sync_copy(st_dst.at[pl.ds(b * STG, 128)],
                            dstb.at[wid, b, pl.ds(o, 128)])
            pltpu.sync_copy(st_w.at[pl.ds(b * STG, 128)],
                            wb.at[wid, b, pl.ds(o, 128)])

        tot = tot + odd
        nbvec = jnp.where(ii == b, jnp.full((16,), tot, jnp.int32), nbvec)
    nbv[...] = nbvec
    pltpu.sync_copy(nbv, nbat.at[wid])

    plsc.subcore_barrier()

    @pl.when(s < 15)
    def _():
        o = pl.multiple_of(s * 3200, 128)
        pltpu.sync_copy(deg_sp.at[pl.ds(o, 3200)], deg2.at[c, pl.ds(o, 3200)])

    @pl.when(s == 15)
    def _():
        pltpu.sync_copy(deg_sp.at[pl.ds(48000, 2000)],
                        deg2.at[c, pl.ds(48000, 2000)])


def _k1(src2d, dst2d, w2d):
    mesh = plsc.VectorSubcoreMesh(core_axis_name="c", subcore_axis_name="s")
    f = functools.partial(
        pl.kernel,
        mesh=mesh,
        compiler_params=pltpu.CompilerParams(
            needs_layout_passes=False, use_tc_tiling_on_sc=False),
        out_type=(
            jax.ShapeDtypeStruct((2, N), jnp.float32),
            jax.ShapeDtypeStruct((NWORK, NB, CAP), jnp.int32),
            jax.ShapeDtypeStruct((NWORK, NB, CAP), jnp.int32),
            jax.ShapeDtypeStruct((NWORK, NB, CAP), jnp.float32),
            jax.ShapeDtypeStruct((NWORK, 16), jnp.int32),
        ),
        scratch_types=[
            pltpu.VMEM((8, 128), jnp.int32),
            pltpu.VMEM((8, 128), jnp.int32),
            pltpu.VMEM((8, 128), jnp.float32),
            pltpu.VMEM((NB * STG,), jnp.int32),
            pltpu.VMEM((NB * STG,), jnp.int32),
            pltpu.VMEM((NB * STG,), jnp.float32),
            pltpu.VMEM((3200,), jnp.float32),
            pltpu.VMEM((16,), jnp.int32),
            pltpu.SemaphoreType.DMA,
            pltpu.VMEM_SHARED((N,), jnp.float32),
        ],
    )(_k1_body)
    return f(src2d, dst2d, w2d)


def _k2_body(deg2, srcb, dstb, wb, nbat, wnb,
             da, db, dv, sidx, didx, dgi, wv, dsv, ddv, wn,
             sidx2, didx2, dgi2, wv2, dsv2, ddv2, wn2, nbv,
             gsem, gsem2, isem, osem, osem2, dinv_sp):
    c = lax.axis_index("c")
    s = lax.axis_index("s")
    wid = s * 2 + c
    ii = lax.iota(jnp.int32, 16)

    def newton(nv):
        half = jnp.full((16,), 0.5, jnp.float32)
        threeh = jnp.full((16,), 1.5, jnp.float32)
        magic = jnp.full((16,), 0x5f3759df, jnp.int32)
        for i in range(nv):
            x = da[pl.ds(16 * i, 16)] + db[pl.ds(16 * i, 16)]
            bits = plsc.bitcast(x, jnp.int32)
            y = plsc.bitcast(magic - lax.shift_right_logical(bits, jnp.full((16,), 1, jnp.int32)),
                             jnp.float32)
            for _ in range(3):
                y = y * (threeh - half * x * y * y)
            dv[pl.ds(16 * i, 16)] = y

    for half in range(2):
        sl = 2 * s + half

        @pl.when(sl < 31)
        def _(sl=sl):
            r0 = pl.multiple_of(sl * 1568, 8)
            pltpu.sync_copy(deg2.at[0, pl.ds(r0, 1568)], da)
            pltpu.sync_copy(deg2.at[1, pl.ds(r0, 1568)], db)

        @pl.when(sl == 31)
        def _():
            pltpu.sync_copy(deg2.at[0, pl.ds(48608, 1392)],
                            da.at[pl.ds(0, 1392)])
            pltpu.sync_copy(deg2.at[1, pl.ds(48608, 1392)],
                            db.at[pl.ds(0, 1392)])

        newton(98)

        @pl.when(sl < 31)
        def _(sl=sl):
            r0 = pl.multiple_of(sl * 1568, 8)
            pltpu.sync_copy(dv, dinv_sp.at[pl.ds(r0, 1568)])

        @pl.when(sl == 31)
        def _():
            pltpu.sync_copy(dv.at[pl.ds(0, 1392)],
                            dinv_sp.at[pl.ds(48608, 1392)])

    plsc.subcore_barrier()

    pltpu.sync_copy(nbat.at[wid], nbv)
    nbvec = nbv[...]
    sx = [sidx, sidx2]
    dx = [didx, didx2]
    gx = [dgi, dgi2]
    wx = [wv, wv2]
    dsx = [dsv, dsv2]
    ddx = [ddv, ddv2]
    wnx = [wn, wn2]
    gs = [gsem, gsem2]
    os_ = [osem, osem2]
    for b in range(NB):
        nb_b = _extract_lane(nbvec, b, ii)
        base = jnp.full((16,), b * BW, jnp.int32)

        def stage_in(k, p, b=b, base=base):
            o = pl.multiple_of(k * 128, 128)
            h1 = pltpu.async_copy(srcb.at[wid, b, pl.ds(o, 128)], sx[p], isem)
            h2 = pltpu.async_copy(dstb.at[wid, b, pl.ds(o, 128)], dx[p], isem)
            h3 = pltpu.async_copy(wb.at[wid, b, pl.ds(o, 128)], wx[p], isem)
            h1.wait()
            h2.wait()
            h3.wait()
            for j in range(8):
                gx[p][pl.ds(16 * j, 16)] = dx[p][pl.ds(16 * j, 16)] + base
            pltpu.async_copy(dinv_sp.at[sx[p]], dsx[p], gs[p])
            pltpu.async_copy(dinv_sp.at[gx[p]], ddx[p], gs[p])

        def finish(k, p, b=b):
            o = pl.multiple_of(k * 128, 128)
            pltpu.make_async_copy(dinv_sp.at[sx[p]], dsx[p], gs[p]).wait()
            pltpu.make_async_copy(dinv_sp.at[gx[p]], ddx[p], gs[p]).wait()
            for j in range(8):
                wnx[p][pl.ds(16 * j, 16)] = (wx[p][pl.ds(16 * j, 16)]
                                             * dsx[p][pl.ds(16 * j, 16)]
                                             * ddx[p][pl.ds(16 * j, 16)])
            pltpu.async_copy(wnx[p], wnb.at[wid, b, pl.ds(o, 128)], os_[p])

        def drain_out(p, b=b):
            pltpu.make_async_copy(
                wnx[p], wnb.at[wid, b, pl.ds(0, 128)], os_[p]).wait()

        @pl.when(nb_b > 0)
        def _(b=b):
            stage_in(0, 0)

        def batch_body(k, _, b=b):
            for p in range(2):
                @pl.when((k & 1) == p)
                def _(p=p, b=b):
                    q = 1 - p

                    @pl.when(k + 1 < nb_b)
                    def _(p=p, q=q, b=b):
                        @pl.when(k >= 1)
                        def _(q=q):
                            drain_out(q)
                        stage_in(k + 1, q)
                    finish(k, p)
            return 0

        lax.fori_loop(0, nb_b, batch_body, 0)

        for p in range(2):
            @pl.when((nb_b >= 2) & (jnp.bitwise_and(nb_b, 1) == p))
            def _(p=p):
                drain_out(p)

            @pl.when((nb_b >= 1) & (jnp.bitwise_and(nb_b - 1, 1) == p))
            def _(p=p):
                drain_out(p)


def _k2(deg2, srcb, dstb, wb, nbat):
    mesh = plsc.VectorSubcoreMesh(core_axis_name="c", subcore_axis_name="s")
    f = functools.partial(
        pl.kernel,
        mesh=mesh,
        compiler_params=pltpu.CompilerParams(
            needs_layout_passes=False, use_tc_tiling_on_sc=False),
        out_type=(jax.ShapeDtypeStruct((NWORK, NB, CAP), jnp.float32),),
        scratch_types=[
            pltpu.VMEM((1568,), jnp.float32),
            pltpu.VMEM((1568,), jnp.float32),
            pltpu.VMEM((1568,), jnp.float32),
            pltpu.VMEM((128,), jnp.int32),
            pltpu.VMEM((128,), jnp.int32),
            pltpu.VMEM((128,), jnp.int32),
            pltpu.VMEM((128,), jnp.float32),
            pltpu.VMEM((128,), jnp.float32),
            pltpu.VMEM((128,), jnp.float32),
            pltpu.VMEM((128,), jnp.float32),
            pltpu.VMEM((128,), jnp.int32),
            pltpu.VMEM((128,), jnp.int32),
            pltpu.VMEM((128,), jnp.int32),
            pltpu.VMEM((128,), jnp.float32),
            pltpu.VMEM((128,), jnp.float32),
            pltpu.VMEM((128,), jnp.float32),
            pltpu.VMEM((128,), jnp.float32),
            pltpu.VMEM((16,), jnp.int32),
            pltpu.SemaphoreType.DMA,
            pltpu.SemaphoreType.DMA,
            pltpu.SemaphoreType.DMA,
            pltpu.SemaphoreType.DMA,
            pltpu.SemaphoreType.DMA,
            pltpu.VMEM_SHARED((N,), jnp.float32),
        ],
    )(_k2_body)
    return f(deg2, srcb, dstb, wb, nbat)


def _make_prop(F):
    FC = F // 16

    def body(xw, srcb, dstb2, wnb, nbat, S,
             sidx, didx, wv, rows, sidx2, didx2, wv2, rows2, nbv, zb,
             gsem, ssem, ssem2, isem, S_sp):
        c = lax.axis_index("c")
        s = lax.axis_index("s")
        ii = lax.iota(jnp.int32, 16)
        zf = jnp.zeros((16,), jnp.float32)

        for r in range(49):
            for j in range(FC):
                zb[r, pl.ds(16 * j, 16)] = zf

        for phase in range(4):
            b = 4 * c + phase
            plsc.subcore_barrier()
            for i in range(8):
                pltpu.sync_copy(zb, S_sp.at[pl.ds((s * 8 + i) * 49, 49)])
            plsc.subcore_barrier()

            for t_off in range(2):
                t = 2 * s + t_off
                pltpu.sync_copy(nbat.at[t], nbv)
                nb_b = _extract_lane(nbv[...], b, ii)

                nb2 = lax.shift_right_logical(nb_b, 1)
                sx = [sidx, sidx2]
                dx = [didx, didx2]
                wx = [wv, wv2]
                rx = [rows, rows2]
                ss = [ssem, ssem2]

                def load_idx(k, p, t=t, b=b):
                    o = pl.multiple_of(k * 256, 256)
                    h1 = pltpu.async_copy(srcb.at[t, b, pl.ds(o, 256)],
                                          sx[p], isem)
                    h2 = pltpu.async_copy(
                        dstb2.at[t, b, pl.ds(pl.multiple_of(2 * k, 2), 2)],
                        dx[p], isem)
                    h3 = pltpu.async_copy(wnb.at[t, b, pl.ds(o, 256)],
                                          wx[p], isem)
                    h1.wait()
                    h2.wait()
                    h3.wait()

                def start_gathers(p):
                    for hh in range(2):
                        pltpu.async_copy(
                            xw.at[sx[p].at[pl.ds(128 * hh, 128)]],
                            rx[p].at[pl.ds(128 * hh, 128)], gsem)

                def wait_gathers(p):
                    for hh in range(2):
                        pltpu.make_async_copy(
                            xw.at[sx[p].at[pl.ds(128 * hh, 128)]],
                            rx[p].at[pl.ds(128 * hh, 128)], gsem).wait()

                def compute(p):
                    def e_outer(j, _2):
                        for i_ in range(8):
                            e = j * 8 + i_
                            wspl = plsc.load_gather(
                                wx[p], [jnp.full((16,), e, jnp.int32)])
                            for fc in range(FC):
                                v = rx[p][e, pl.ds(16 * fc, 16)].reshape(
                                    (16,)) * wspl
                                rx[p][e, pl.ds(16 * fc, 16)] = v
                        return 0

                    lax.fori_loop(0, 32, e_outer, 0)

                def start_scatters(p):
                    for hh in range(2):
                        pltpu.async_copy(rx[p].at[pl.ds(128 * hh, 128)],
                                         S_sp.at[dx[p].at[hh]], ss[p],
                                         add=True)

                def drain_scatter(p):
                    for hh in range(2):
                        pltpu.make_async_copy(
                            rx[p].at[pl.ds(128 * hh, 128)],
                            S_sp.at[dx[p].at[hh]], ss[p]).wait()

                @pl.when(nb2 > 0)
                def _(t=t, b=b):
                    load_idx(0, 0)
                    start_gathers(0)

                def batch_body(k, _, t=t, b=b):
                    for p in range(2):
                        @pl.when((k & 1) == p)
                        def _(p=p, t=t, b=b):
                            q = 1 - p
                            wait_gathers(p)

                            @pl.when(k + 1 < nb2)
                            def _(p=p, q=q, t=t, b=b):
                                @pl.when(k >= 1)
                                def _(q=q):
                                    drain_scatter(q)
                                load_idx(k + 1, q)
                                start_gathers(q)

                            compute(p)
                            start_scatters(p)
                    return 0

                lax.fori_loop(0, nb2, batch_body, 0)

                for p in range(2):
                    @pl.when((nb2 >= 2) & (jnp.bitwise_and(nb2, 1) == p))
                    def _(p=p):
                        drain_scatter(p)

                    @pl.when((nb2 >= 1)
                             & (jnp.bitwise_and(nb2 - 1, 1) == p))
                    def _(p=p):
                        drain_scatter(p)

            plsc.subcore_barrier()
            r0 = 392 * s

            @pl.when(s < 15)
            def _(b=b, r0=r0):
                pltpu.sync_copy(S_sp.at[pl.ds(r0, 392)],
                                S.at[pl.ds(b * BW + r0, 392)])

            @pl.when(s == 15)
            def _(b=b):
                pltpu.sync_copy(S_sp.at[pl.ds(5880, 370)],
                                S.at[pl.ds(b * BW + 5880, 370)])

    mesh = plsc.VectorSubcoreMesh(core_axis_name="c", subcore_axis_name="s")

    def run(xw, srcb, dstb, wnb, nbat):
        f = functools.partial(
            pl.kernel,
            mesh=mesh,
            compiler_params=pltpu.CompilerParams(
                needs_layout_passes=False, use_tc_tiling_on_sc=False),
            out_type=(jax.ShapeDtypeStruct((N, F), jnp.float32),),
            scratch_types=[
                pltpu.VMEM((256,), jnp.int32),
                pltpu.VMEM((2, 128), jnp.int32),
                pltpu.VMEM((256,), jnp.float32),
                pltpu.VMEM((256, F), jnp.float32),
                pltpu.VMEM((256,), jnp.int32),
                pltpu.VMEM((2, 128), jnp.int32),
                pltpu.VMEM((256,), jnp.float32),
                pltpu.VMEM((256, F), jnp.float32),
                pltpu.VMEM((16,), jnp.int32),
                pltpu.VMEM((49, F), jnp.float32),
                pltpu.SemaphoreType.DMA,
                pltpu.SemaphoreType.DMA,
                pltpu.SemaphoreType.DMA,
                pltpu.SemaphoreType.DMA,
                pltpu.VMEM_SHARED((BP, F), jnp.float32),
            ],
        )(body)
        res = f(xw, srcb, dstb.reshape(NWORK, NB, CAP // 128, 128),
                wnb, nbat)
        return res[0] if isinstance(res, (tuple, list)) else res

    return run


def _pool_body(S3, ac, batch, psum2, pcnt2,
               tb, bidx, tb48, bidx48, ones112, ones48, acv, zb, zc,
               gsem, psum_sp, pcnt_sp):
    c = lax.axis_index("c")
    s = lax.axis_index("s")
    wid = s * 2 + c
    zf = jnp.zeros((16,), jnp.float32)
    onef = jnp.full((16,), 1.0, jnp.float32)

    for r in range(64):
        for j in range(4):
            zb[r, pl.ds(16 * j, 16)] = zf
    for i in range(4):
        zc[pl.ds(16 * i, 16)] = zf
    for i in range(7):
        ones112[pl.ds(16 * i, 16)] = onef
    for i in range(3):
        ones48[pl.ds(16 * i, 16)] = onef

    pltpu.sync_copy(ac, acv)
    a_l = [acv[0, pl.ds(16 * j, 16)].reshape((16,)) for j in range(4)]
    c_l = [acv[1, pl.ds(16 * j, 16)].reshape((16,)) for j in range(4)]

    pltpu.sync_copy(zb, psum_sp.at[pl.ds(s * 64, 64)])
    pltpu.sync_copy(zc, pcnt_sp.at[pl.ds(s * 64, 64)])
    plsc.subcore_barrier()

    def do_chunk(buf, bx, ones, nrows, r0):
        pltpu.sync_copy(S3.at[pl.ds(r0, nrows)], buf)
        pltpu.sync_copy(batch.at[pl.ds(r0, nrows)], bx)

        def row_body(r8, _):
            for i_ in range(8):
                r = r8 * 8 + i_
                for j in range(4):
                    v = buf[r, pl.ds(16 * j, 16)].reshape((16,))
                    z = jnp.maximum(v * a_l[j] + c_l[j], 0.0)
                    buf[r, pl.ds(16 * j, 16)] = z
            return 0

        lax.fori_loop(0, nrows // 8, row_body, 0)
        pltpu.async_copy(buf, psum_sp.at[bx], gsem, add=True).wait()
        pltpu.async_copy(ones, pcnt_sp.at[bx], gsem, add=True).wait()

    @pl.when(wid < 31)
    def _():
        def chunk_loop(k, _):
            r0 = pl.multiple_of(wid * 1568 + k * 112, 8)
            do_chunk(tb, bidx, ones112, 112, r0)
            return 0
        lax.fori_loop(0, 14, chunk_loop, 0)

    @pl.when(wid == 31)
    def _():
        def chunk_loop(k, _):
            r0 = pl.multiple_of(48608 + k * 112, 8)
            do_chunk(tb, bidx, ones112, 112, r0)
            return 0
        lax.fori_loop(0, 12, chunk_loop, 0)
        do_chunk(tb48, bidx48, ones48, 48, 49952)

    plsc.subcore_barrier()
    pltpu.sync_copy(psum_sp.at[pl.ds(s * 64, 64)],
                    psum2.at[c, pl.ds(s * 64, 64)])
    pltpu.sync_copy(pcnt_sp.at[pl.ds(s * 64, 64)],
                    pcnt2.at[c, pl.ds(s * 64, 64)])


def _pool(S3, ac, batch):
    mesh = plsc.VectorSubcoreMesh(core_axis_name="c", subcore_axis_name="s")
    f = functools.partial(
        pl.kernel,
        mesh=mesh,
        compiler_params=pltpu.CompilerParams(
            needs_layout_passes=False, use_tc_tiling_on_sc=False),
        out_type=(
            jax.ShapeDtypeStruct((2, NUM_GRAPHS, 64), jnp.float32),
            jax.ShapeDtypeStruct((2, NUM_GRAPHS), jnp.float32),
        ),
        scratch_types=[
            pltpu.VMEM((112, 64), jnp.float32),
            pltpu.VMEM((112,), jnp.int32),
            pltpu.VMEM((48, 64), jnp.float32),
            pltpu.VMEM((48,), jnp.int32),
            pltpu.VMEM((112,), jnp.float32),
            pltpu.VMEM((48,), jnp.float32),
            pltpu.VMEM((2, 64), jnp.float32),
            pltpu.VMEM((64, 64), jnp.float32),
            pltpu.VMEM((64,), jnp.float32),
            pltpu.SemaphoreType.DMA,
            pltpu.VMEM_SHARED((NUM_GRAPHS, 64), jnp.float32),
            pltpu.VMEM_SHARED((NUM_GRAPHS,), jnp.float32),
        ],
    )(_pool_body)
    return f(S3, ac, batch)


def _tc1_kernel(x_ref, wc_ref, cb_ref, w1_ref, out_ref):
    h = jnp.maximum(
        jnp.dot(x_ref[...], wc_ref[...],
                preferred_element_type=jnp.float32) + cb_ref[...], 0.0)
    out_ref[...] = jnp.dot(h, w1_ref[...], preferred_element_type=jnp.float32)


def _tc1(x, wc, cb, w1):
    return pl.pallas_call(
        _tc1_kernel,
        grid=(NBLK,),
        in_specs=[
            pl.BlockSpec((BM, 200), lambda i: (i, 0)),
            pl.BlockSpec((200, 192), lambda i: (0, 0)),
            pl.BlockSpec((1, 192), lambda i: (0, 0)),
            pl.BlockSpec((192, 128), lambda i: (0, 0)),
        ],
        out_specs=pl.BlockSpec((BM, 128), lambda i: (i, 0)),
        out_shape=jax.ShapeDtypeStruct((N, 128), jnp.float32),
    )(x, wc, cb, w1)


def _make_tcb(F):
    def kern(s_ref, b_ref, g_ref, be_ref, out_ref, acc_ref):
        i = pl.program_id(0)

        @pl.when(i == 0)
        def _():
            acc_ref[...] = jnp.zeros_like(acc_ref)

        t = s_ref[...] + b_ref[...]
        acc_ref[0:1, :] += jnp.sum(t, axis=0, keepdims=True)
        acc_ref[1:2, :] += jnp.sum(t * t, axis=0, keepdims=True)

        @pl.when(i == NBLK - 1)
        def _():
            mu = acc_ref[0:1, :] / N
            var = acc_ref[1:2, :] / N - mu * mu
            rs = lax.rsqrt(var + 1e-5)
            a = rs * g_ref[...]
            out_ref[0:1, :] = a
            out_ref[1:2, :] = (b_ref[...] - mu) * a + be_ref[...]

    def run(S, b, g, be):
        return pl.pallas_call(
            kern,
            grid=(NBLK,),
            in_specs=[
                pl.BlockSpec((BM, F), lambda i: (i, 0)),
                pl.BlockSpec((1, F), lambda i: (0, 0)),
                pl.BlockSpec((1, F), lambda i: (0, 0)),
                pl.BlockSpec((1, F), lambda i: (0, 0)),
            ],
            out_specs=pl.BlockSpec((2, F), lambda i: (0, 0)),
            out_shape=jax.ShapeDtypeStruct((2, F), jnp.float32),
            scratch_shapes=[pltpu.VMEM((2, F), jnp.float32)],
        )(S, b.reshape(1, F), g.reshape(1, F), be.reshape(1, F))

    return run


def _make_tc23(F, FO):
    def kern(s_ref, ac_ref, w_ref, out_ref):
        z = jnp.maximum(s_ref[...] * ac_ref[0:1, :] + ac_ref[1:2, :], 0.0)
        out_ref[...] = jnp.dot(z, w_ref[...],
                               preferred_element_type=jnp.float32)

    def run(S, ac, w):
        return pl.pallas_call(
            kern,
            grid=(NBLK,),
            in_specs=[
                pl.BlockSpec((BM, F), lambda i: (i, 0)),
                pl.BlockSpec((2, F), lambda i: (0, 0)),
                pl.BlockSpec((F, FO), lambda i: (0, 0)),
            ],
            out_specs=pl.BlockSpec((BM, FO), lambda i: (i, 0)),
            out_shape=jax.ShapeDtypeStruct((N, FO), jnp.float32),
        )(S, ac, w)

    return run


def _tcf_kernel(ps_ref, pc_ref, fw_ref, fb_ref, out_ref):
    ps = ps_ref[0] + ps_ref[1]
    cnt = pc_ref[0] + pc_ref[1]
    pooled = ps / jnp.maximum(cnt, 1.0)[:, None]
    logits = jnp.dot(pooled, fw_ref[...],
                     preferred_element_type=jnp.float32) + fb_ref[...]
    col = lax.broadcasted_iota(jnp.int32, logits.shape, 1)
    logits = jnp.where(col < 4, logits, -1e30)
    m = jnp.max(logits, axis=1, keepdims=True)
    sh = logits - m
    lse = jnp.log(jnp.sum(jnp.exp(sh), axis=1, keepdims=True))
    out_ref[...] = sh - lse


def _tcf(psum2, pcnt2, fw, fb):
    return pl.pallas_call(
        _tcf_kernel,
        out_shape=jax.ShapeDtypeStruct((NUM_GRAPHS, 128), jnp.float32),
    )(psum2, pcnt2, fw, fb)


_DBG = "full"


def kernel(x, edge_index, edge_attr, batch, conv_w, conv_b, W1, b1, W2, b2, W3, b3, g1, be1, g2, be2, g3, be3, fc_w, fc_b):
    loop = jnp.arange(N, dtype=jnp.int32)
    pad = ROWS * 128 - EA
    srcA = jnp.concatenate(
        [edge_index[0], loop, jnp.zeros((pad,), jnp.int32)]).reshape(ROWS, 128)
    dstA = jnp.concatenate(
        [edge_index[1], loop, jnp.zeros((pad,), jnp.int32)]).reshape(ROWS, 128)
    wA = jnp.concatenate(
        [edge_attr, jnp.ones((N,), jnp.float32),
         jnp.zeros((pad,), jnp.float32)]).reshape(ROWS, 128)

    deg2, srcb, dstb, wb, nbat = _k1(srcA, dstA, wA)
    wnb = _k2(deg2, srcb, dstb, wb, nbat)
    if isinstance(wnb, (tuple, list)):
        wnb = wnb[0]

    wct = conv_w[:, 0, :].T
    wc = jnp.zeros((200, 32, 6), jnp.float32)
    for r in range(6):
        wc = wc.at[20 * r:20 * r + 100, :, r].set(wct)
    wc = wc.reshape(200, 192)
    cb = jnp.repeat(conv_b, 6).reshape(1, 192)

    p128 = _make_prop(128)
    p64 = _make_prop(64)
    tcb128 = _make_tcb(128)
    tcb64 = _make_tcb(64)

    def prop_jnp(q):
        cnt = nbat[:, :NB] * 128
        ar = jnp.arange(CAP)[None, None, :]
        valid = ar < cnt[:, :, None]
        w_eff = jnp.where(valid, wnb, 0.0).reshape(-1)
        src_eff = jnp.where(valid, srcb, 0).reshape(-1)
        dg = jnp.clip(dstb, 0, BW - 1) + (
            jnp.arange(NB, dtype=jnp.int32)[None, :, None] * BW)
        dst_eff = jnp.where(valid, dg, 0).reshape(-1)
        msg = q[src_eff] * w_eff[:, None]
        return jnp.zeros_like(q).at[dst_eff].add(msg)

    def bn_jnp(t, g, be):
        mu = jnp.mean(t, axis=0)
        va = jnp.var(t, axis=0)
        return (t - mu) * lax.rsqrt(va + 1e-5) * g + be

    if _DBG in ("k12", "prop"):
        prop = prop_jnp if _DBG == "k12" else (
            lambda q, F=None: None)
        xw1j = jnp.maximum(x @ wc + cb, 0.0) @ W1
        if _DBG == "k12":
            S1j = prop_jnp(xw1j)
        else:
            S1j = p128(xw1j, srcb, dstb, wnb, nbat)
        h = jnp.maximum(bn_jnp(S1j + b1, g1, be1), 0.0)
        xw2j = h @ W2
        S2j = prop_jnp(xw2j) if _DBG == "k12" else p128(
            xw2j, srcb, dstb, wnb, nbat)
        h = jnp.maximum(bn_jnp(S2j + b2, g2, be2), 0.0)
        xw3j = h @ W3
        S3j = prop_jnp(xw3j) if _DBG == "k12" else p64(
            xw3j, srcb, dstb, wnb, nbat)
        h = jnp.maximum(bn_jnp(S3j + b3, g3, be3), 0.0)
        sums = jax.ops.segment_sum(h, batch, num_segments=NUM_GRAPHS)
        cntg = jax.ops.segment_sum(jnp.ones((N,), jnp.float32), batch,
                                   num_segments=NUM_GRAPHS)
        pooled = sums / jnp.maximum(cntg, 1.0)[:, None]
        logits = pooled @ fc_w + fc_b
        return jax.nn.log_softmax(logits, axis=1)

    xw1 = _tc1(x, wc, cb, W1)
    S1 = prop_jnp(xw1) if _DBG == "tc" else p128(xw1, srcb, dstb, wnb, nbat)
    ac1 = tcb128(S1, b1, g1, be1)
    xw2 = _make_tc23(128, 128)(S1, ac1, W2)
    S2 = prop_jnp(xw2) if _DBG == "tc" else p128(xw2, srcb, dstb, wnb, nbat)
    ac2 = tcb128(S2, b2, g2, be2)
    xw3 = _make_tc23(128, 64)(S2, ac2, W3)
    S3 = prop_jnp(xw3) if _DBG == "tc" else p64(xw3, srcb, dstb, wnb, nbat)
    ac3 = tcb64(S3, b3, g3, be3)
    psum2, pcnt2 = _pool(S3, ac3, batch)

    fw = jnp.zeros((64, 128), jnp.float32).at[:, :4].set(fc_w)
    fb = jnp.zeros((1, 128), jnp.float32).at[0, :4].set(fc_b)
    out = _tcf(psum2, pcnt2, fw, fb)
    return out[:, :4]
